# Initial kernel scaffold; baseline (speedup 1.0000x reference)
import jax
import jax.numpy as jnp
from jax.experimental import pallas as pl


def kernel(h, pos, edge_index, h_init, W_e1, b_e1, W_e2, b_e2, W_c1, b_c1, W_c2, W_h1, b_h1, W_h2, b_h2):
    raise NotImplementedError("write your pallas kernel here")


# speedup vs baseline: 3.7953x; 3.7953x over previous
import functools

import jax
import jax.numpy as jnp
from jax import lax
from jax.experimental import pallas as pl
from jax.experimental.pallas import tpu as pltpu
from jax.experimental.pallas import tpu_sc as plsc

N = 10000
E = 320000
D = 128
H = 128

NPAD = 10240
EPAD = 327680
NC = 2
NS = 16
NW = NC * NS
EW = EPAD // NW
CHUNK = 128
NCHUNK = EW // CHUNK
ROWS_PER_TILE = NPAD // NS

NB = 1024
EB = 2048


def _silu(x):
    return x * jax.nn.sigmoid(x)


def _prep_body(h_ref, hi_ref, w1d_ref, w1s_ref, b1_ref, hp_ref, ha_ref, hb_ref):
    hp = h_ref[...] + hi_ref[...]
    hp_ref[...] = hp
    ha_ref[...] = jnp.dot(hp, w1d_ref[...], preferred_element_type=jnp.float32) + b1_ref[...]
    hb_ref[...] = jnp.dot(hp, w1s_ref[...], preferred_element_type=jnp.float32)


def _prep(h_pad, hi_pad, w1d, w1s, b1):
    grid = (NPAD // NB,)
    blk = pl.BlockSpec((NB, D), lambda i: (i, 0))
    wblk = pl.BlockSpec((D, H), lambda i: (0, 0))
    bblk = pl.BlockSpec((1, H), lambda i: (0, 0))
    return pl.pallas_call(
        _prep_body,
        grid=grid,
        in_specs=[blk, blk, wblk, wblk, bblk],
        out_specs=[blk, pl.BlockSpec((NB, H), lambda i: (i, 0)),
                   pl.BlockSpec((NB, H), lambda i: (i, 0))],
        out_shape=[jax.ShapeDtypeStruct((NPAD, D), jnp.float32),
                   jax.ShapeDtypeStruct((NPAD, H), jnp.float32),
                   jax.ShapeDtypeStruct((NPAD, H), jnp.float32)],
    )(h_pad, hi_pad, w1d, w1s, b1)


def _gather_body(ha_hbm, hb_hbm, pos_hbm, src_hbm, dst_hbm,
                 qa_out, qb_out, geom_out,
                 sidx_v, didx_v, qa_v, qb_v, pos_v, geom_v, sem):
    wid = lax.axis_index("s") * NC + lax.axis_index("c")
    pltpu.sync_copy(pos_hbm, pos_v)
    z16 = jnp.zeros((16,), jnp.float32)
    for c in range(3, 8):
        for g in range(CHUNK // 16):
            geom_v[c, pl.ds(g * 16, 16)] = z16

    def body(i, _):
        base = wid * EW + i * CHUNK
        pltpu.sync_copy(src_hbm.at[pl.ds(base, CHUNK)], sidx_v)
        pltpu.sync_copy(dst_hbm.at[pl.ds(base, CHUNK)], didx_v)
        c1 = pltpu.async_copy(ha_hbm.at[didx_v], qa_v, sem)
        c2 = pltpu.async_copy(hb_hbm.at[sidx_v], qb_v, sem)

        def grp(g, carry):
            s16 = sidx_v[pl.ds(g * 16, 16)] * 4
            d16 = didx_v[pl.ds(g * 16, 16)] * 4
            for c in range(3):
                psc = plsc.load_gather(pos_v, [s16 + c])
                pdc = plsc.load_gather(pos_v, [d16 + c])
                geom_v[c, pl.ds(g * 16, 16)] = pdc - psc
            return carry

        lax.fori_loop(0, CHUNK // 16, grp, None)
        c1.wait(); c2.wait()
        pltpu.sync_copy(qa_v, qa_out.at[pl.ds(base, CHUNK)])
        pltpu.sync_copy(qb_v, qb_out.at[pl.ds(base, CHUNK)])
        pltpu.sync_copy(geom_v, geom_out.at[:, pl.ds(base, CHUNK)])
        return _

    lax.fori_loop(0, NCHUNK, body, None)


def _gather(ha, hb, pos4, src, dst):
    mesh = plsc.VectorSubcoreMesh(core_axis_name="c", subcore_axis_name="s")
    fn = pl.kernel(
        _gather_body,
        out_type=[jax.ShapeDtypeStruct((EPAD, H), jnp.float32),
                  jax.ShapeDtypeStruct((EPAD, H), jnp.float32),
                  jax.ShapeDtypeStruct((8, EPAD), jnp.float32)],
        mesh=mesh,
        scratch_types=[pltpu.VMEM((CHUNK,), jnp.int32),
                       pltpu.VMEM((CHUNK,), jnp.int32),
                       pltpu.VMEM((CHUNK, H), jnp.float32),
                       pltpu.VMEM((CHUNK, H), jnp.float32),
                       pltpu.VMEM((NPAD * 4,), jnp.float32),
                       pltpu.VMEM((8, CHUNK), jnp.float32),
                       pltpu.SemaphoreType.DMA],
        compiler_params=pltpu.CompilerParams(needs_layout_passes=False),
    )
    return fn(ha, hb, pos4, src, dst)


def _mlp_body(qa_ref, qb_ref, geom_ref, w1e_ref, we2_ref, be2_ref,
              wc1_ref, bc1_ref, wc2_ref, m2_ref, rw_ref):
    rel = jnp.transpose(geom_ref[...])
    dist2 = jnp.sum(rel * rel, axis=1, keepdims=True)
    m1 = _silu(qa_ref[...] + qb_ref[...] + dist2 * w1e_ref[...])
    m2 = _silu(jnp.dot(m1, we2_ref[...], preferred_element_type=jnp.float32) + be2_ref[...])
    cw = jnp.dot(_silu(jnp.dot(m2, wc1_ref[...], preferred_element_type=jnp.float32) + bc1_ref[...]),
                 wc2_ref[...], preferred_element_type=jnp.float32)
    m2_ref[...] = m2
    sub = lax.broadcasted_iota(jnp.int32, (8, EB), 0)
    rw_ref[...] = jnp.where(sub == 3, 1.0, geom_ref[...] * jnp.transpose(cw))


def _mlp(qa, qb, geom, w1e, we2, be2, wc1, bc1, wc2):
    grid = (EPAD // EB,)
    eblk = pl.BlockSpec((EB, H), lambda i: (i, 0))
    gblk = pl.BlockSpec((8, EB), lambda i: (0, i))
    full = lambda shape: pl.BlockSpec(shape, lambda i: tuple(0 for _ in shape))
    return pl.pallas_call(
        _mlp_body,
        grid=grid,
        in_specs=[eblk, eblk, gblk,
                  full((1, H)), full((H, H)), full((1, H)),
                  full((H, H)), full((1, H)), full((H, 1))],
        out_specs=[eblk, gblk],
        out_shape=[jax.ShapeDtypeStruct((EPAD, H), jnp.float32),
                   jax.ShapeDtypeStruct((8, EPAD), jnp.float32)],
    )(qa, qb, geom, w1e, we2, be2, wc1, bc1, wc2)


TPT = EPAD // NS
NCH2 = TPT // CHUNK


def _scatter_body(m2_hbm, rw_hbm, dst_hbm, z128_hbm,
                  magg_out, posacc_out,
                  acc, didx_v, m2_v, rw_tv, rw128_v, sem):
    cid = lax.axis_index("c")
    sid = lax.axis_index("s")
    rbase = sid * ROWS_PER_TILE
    pltpu.sync_copy(z128_hbm.at[pl.ds(rbase, ROWS_PER_TILE)],
                    acc.at[pl.ds(rbase, ROWS_PER_TILE)])
    plsc.subcore_barrier()

    @pl.when(cid == 1)
    def _m2_loop():
        def body(i, carry):
            base = sid * TPT + i * CHUNK
            pltpu.sync_copy(dst_hbm.at[pl.ds(base, CHUNK)], didx_v)
            pltpu.async_copy(m2_hbm.at[pl.ds(base, CHUNK)], m2_v, sem).wait()
            pltpu.sync_copy(m2_v, acc.at[didx_v], add=True)
            return carry
        lax.fori_loop(0, NCH2, body, None)

    @pl.when(cid == 0)
    def _rw_loop():
        pltpu.sync_copy(z128_hbm.at[pl.ds(0, CHUNK)], rw128_v)
        lane16 = lax.broadcasted_iota(jnp.int32, (16,), 0)

        def body(i, carry):
            base = sid * TPT + i * CHUNK
            pltpu.sync_copy(dst_hbm.at[pl.ds(base, CHUNK)], didx_v)
            pltpu.sync_copy(rw_hbm.at[:, pl.ds(base, CHUNK)], rw_tv)

            def grp(g, c2):
                e16 = g * 16 + lane16
                for c in range(4):
                    plsc.store_scatter(rw128_v, [e16, jnp.full((16,), c, jnp.int32)],
                                       rw_tv[c, pl.ds(g * 16, 16)])
                return c2
            lax.fori_loop(0, CHUNK // 16, grp, None)
            pltpu.sync_copy(rw128_v, acc.at[didx_v], add=True)
            return carry
        lax.fori_loop(0, NCH2, body, None)

    plsc.subcore_barrier()

    @pl.when(cid == 1)
    def _dump_m():
        pltpu.sync_copy(acc.at[pl.ds(rbase, ROWS_PER_TILE)],
                        magg_out.at[pl.ds(rbase, ROWS_PER_TILE)])

    @pl.when(cid == 0)
    def _dump_p():
        pltpu.sync_copy(acc.at[pl.ds(rbase, ROWS_PER_TILE)],
                        posacc_out.at[pl.ds(rbase, ROWS_PER_TILE)])


def _scatter(m2, rw, dst, z128):
    mesh = plsc.VectorSubcoreMesh(core_axis_name="c", subcore_axis_name="s")
    fn = pl.kernel(
        _scatter_body,
        out_type=[jax.ShapeDtypeStruct((NPAD, H), jnp.float32),
                  jax.ShapeDtypeStruct((NPAD, H), jnp.float32)],
        mesh=mesh,
        scratch_types=[pltpu.VMEM_SHARED((NPAD, H), jnp.float32),
                       pltpu.VMEM((CHUNK,), jnp.int32),
                       pltpu.VMEM((CHUNK, H), jnp.float32),
                       pltpu.VMEM((8, CHUNK), jnp.float32),
                       pltpu.VMEM((CHUNK, H), jnp.float32),
                       pltpu.SemaphoreType.DMA],
        compiler_params=pltpu.CompilerParams(needs_layout_passes=False),
    )
    return fn(m2, rw, dst, z128)


def _final_body(hp_ref, magg_ref, pacc_ref, pos_ref, wh1a_ref, wh1b_ref,
                bh1_ref, wh2_ref, bh2_ref, hout_ref, pout_ref):
    hp = hp_ref[...]
    magg = magg_ref[...]
    pacc = pacc_ref[...]
    t = _silu(jnp.dot(hp, wh1a_ref[...], preferred_element_type=jnp.float32)
              + jnp.dot(magg, wh1b_ref[...], preferred_element_type=jnp.float32)
              + bh1_ref[...])
    hout_ref[...] = hp + jnp.dot(t, wh2_ref[...], preferred_element_type=jnp.float32) + bh2_ref[...]
    lane = lax.broadcasted_iota(jnp.int32, (NB, H), 1)
    cnt = jnp.sum(jnp.where(lane == 3, pacc, 0.0), axis=1, keepdims=True)
    upd = jnp.where(lane < 3, pacc, 0.0) / jnp.maximum(cnt, 1.0)
    pout_ref[...] = pos_ref[...] + upd


def _final(hp, magg, pacc, pos128, wh1a, wh1b, bh1, wh2, bh2):
    grid = (NPAD // NB,)
    nblk = pl.BlockSpec((NB, D), lambda i: (i, 0))
    full = lambda shape: pl.BlockSpec(shape, lambda i: tuple(0 for _ in shape))
    return pl.pallas_call(
        _final_body,
        grid=grid,
        in_specs=[nblk, nblk, nblk, nblk,
                  full((D, H)), full((H, H)), full((1, H)),
                  full((H, D)), full((1, D))],
        out_specs=[nblk, nblk],
        out_shape=[jax.ShapeDtypeStruct((NPAD, D), jnp.float32),
                   jax.ShapeDtypeStruct((NPAD, D), jnp.float32)],
    )(hp, magg, pacc, pos128, wh1a, wh1b, bh1, wh2, bh2)


@jax.jit
def kernel(h, pos, edge_index, h_init, W_e1, b_e1, W_e2, b_e2, W_c1, b_c1,
           W_c2, W_h1, b_h1, W_h2, b_h2):
    h_pad = jnp.pad(h, ((0, NPAD - N), (0, 0)))
    hi_pad = jnp.pad(h_init, ((0, NPAD - N), (0, 0)))
    pos128 = jnp.pad(pos, ((0, NPAD - N), (0, D - 3)))
    pos4 = jnp.pad(pos, ((0, NPAD - N), (0, 1))).reshape(-1)
    src = jnp.pad(edge_index[0], (0, EPAD - E))
    dst = jnp.pad(edge_index[1], (0, EPAD - E), constant_values=N)

    w1d = W_e1[:D]
    w1s = W_e1[D:2 * D]
    w1e = W_e1[2 * D:]
    b1 = b_e1.reshape(1, H)
    be2 = b_e2.reshape(1, H)
    bc1 = b_c1.reshape(1, H)
    wh1a = W_h1[:D]
    wh1b = W_h1[D:]
    bh1 = b_h1.reshape(1, H)
    bh2 = b_h2.reshape(1, D)

    hp, ha, hb = _prep(h_pad, hi_pad, w1d, w1s, b1)
    qa, qb, geom = _gather(ha, hb, pos4, src, dst)
    m2, rw = _mlp(qa, qb, geom, w1e, W_e2, be2, W_c1, bc1, W_c2)
    z128 = jnp.zeros((NPAD, H), jnp.float32)
    magg, pacc = _scatter(m2, rw, dst, z128)
    h_out, pos_out = _final(hp, magg, pacc, pos128, wh1a, wh1b, bh1, W_h2, bh2)
    return (h_out[:N], pos_out[:N, :3])


# speedup vs baseline: 4.3597x; 1.1487x over previous
import functools

import jax
import jax.numpy as jnp
from jax import lax
from jax.experimental import pallas as pl
from jax.experimental.pallas import tpu as pltpu
from jax.experimental.pallas import tpu_sc as plsc

N = 10000
E = 320000
D = 128
H = 128

NPAD = 10240
EPAD = 327680
NC = 2
NS = 16
NW = NC * NS
EW = EPAD // NW
CHUNK = 128
NCHUNK = EW // CHUNK
ROWS_PER_TILE = NPAD // NS

NB = 1024
EB = 2048


def _silu(x):
    return x * jax.nn.sigmoid(x)


def _prep_body(h_ref, hi_ref, w1d_ref, w1s_ref, b1_ref, hp_ref, ha_ref, hb_ref):
    hp = h_ref[...] + hi_ref[...]
    hp_ref[...] = hp
    ha_ref[...] = jnp.dot(hp, w1d_ref[...], preferred_element_type=jnp.float32) + b1_ref[...]
    hb_ref[...] = jnp.dot(hp, w1s_ref[...], preferred_element_type=jnp.float32)


def _prep(h_pad, hi_pad, w1d, w1s, b1):
    grid = (NPAD // NB,)
    blk = pl.BlockSpec((NB, D), lambda i: (i, 0))
    wblk = pl.BlockSpec((D, H), lambda i: (0, 0))
    bblk = pl.BlockSpec((1, H), lambda i: (0, 0))
    return pl.pallas_call(
        _prep_body,
        grid=grid,
        in_specs=[blk, blk, wblk, wblk, bblk],
        out_specs=[blk, pl.BlockSpec((NB, H), lambda i: (i, 0)),
                   pl.BlockSpec((NB, H), lambda i: (i, 0))],
        out_shape=[jax.ShapeDtypeStruct((NPAD, D), jnp.float32),
                   jax.ShapeDtypeStruct((NPAD, H), jnp.float32),
                   jax.ShapeDtypeStruct((NPAD, H), jnp.float32)],
    )(h_pad, hi_pad, w1d, w1s, b1)


def _gather_body(ha_hbm, hb_hbm, pos_hbm, src_hbm, dst_hbm,
                 qa_out, qb_out, geom_out,
                 sidx_v, didx_v, qa_v, qb_v, pos_v, geom_v,
                 isem0, isem1, gsem0, gsem1, wsem0, wsem1):
    wid = lax.axis_index("s") * NC + lax.axis_index("c")
    isem = (isem0, isem1)
    gsem = (gsem0, gsem1)
    wsem = (wsem0, wsem1)
    pltpu.sync_copy(pos_hbm, pos_v)
    z16 = jnp.zeros((16,), jnp.float32)
    for b in range(2):
        for c in range(3, 8):
            for g in range(CHUNK // 16):
                geom_v[b, c, pl.ds(g * 16, 16)] = z16

    def issue_idx(k, b):
        base = wid * EW + k * CHUNK
        pltpu.async_copy(src_hbm.at[pl.ds(base, CHUNK)], sidx_v.at[b], isem[b])
        pltpu.async_copy(dst_hbm.at[pl.ds(base, CHUNK)], didx_v.at[b], isem[b])

    def wait_idx(b):
        pltpu.make_async_copy(src_hbm.at[pl.ds(0, CHUNK)], sidx_v.at[b], isem[b]).wait()
        pltpu.make_async_copy(dst_hbm.at[pl.ds(0, CHUNK)], didx_v.at[b], isem[b]).wait()

    def fire_gathers(b):
        pltpu.async_copy(ha_hbm.at[didx_v.at[b]], qa_v.at[b], gsem[b])
        pltpu.async_copy(hb_hbm.at[sidx_v.at[b]], qb_v.at[b], gsem[b])

    def wait_gathers(b):
        pltpu.make_async_copy(ha_hbm.at[pl.ds(0, CHUNK)], qa_v.at[b], gsem[b]).wait()
        pltpu.make_async_copy(hb_hbm.at[pl.ds(0, CHUNK)], qb_v.at[b], gsem[b]).wait()

    def compute_geom(b):
        def grp(g, carry):
            s16 = sidx_v[b, pl.ds(g * 16, 16)] * 4
            d16 = didx_v[b, pl.ds(g * 16, 16)] * 4
            for c in range(3):
                psc = plsc.load_gather(pos_v, [s16 + c])
                pdc = plsc.load_gather(pos_v, [d16 + c])
                geom_v[b, c, pl.ds(g * 16, 16)] = pdc - psc
            return carry
        lax.fori_loop(0, CHUNK // 16, grp, None)

    def fire_writebacks(k, b):
        base = wid * EW + k * CHUNK
        pltpu.async_copy(qa_v.at[b], qa_out.at[pl.ds(base, CHUNK)], wsem[b])
        pltpu.async_copy(qb_v.at[b], qb_out.at[pl.ds(base, CHUNK)], wsem[b])
        pltpu.async_copy(geom_v.at[b], geom_out.at[:, pl.ds(base, CHUNK)], wsem[b])

    def wait_writebacks(b):
        pltpu.make_async_copy(qa_v.at[b], qa_out.at[pl.ds(0, CHUNK)], wsem[b]).wait()
        pltpu.make_async_copy(qb_v.at[b], qb_out.at[pl.ds(0, CHUNK)], wsem[b]).wait()
        pltpu.make_async_copy(geom_v.at[b], geom_out.at[:, pl.ds(0, CHUNK)], wsem[b]).wait()

    issue_idx(0, 0)
    issue_idx(1, 1)
    wait_idx(0)
    fire_gathers(0)
    compute_geom(0)

    def body(j, carry):
        for b in range(2):
            k = 2 * j + b
            b1 = 1 - b

            @pl.when(k + 1 < NCHUNK)
            def _start_next():
                wait_idx(b1)

                @pl.when(k >= 1)
                def _free_slot():
                    wait_writebacks(b1)
                fire_gathers(b1)
                compute_geom(b1)

            wait_gathers(b)
            fire_writebacks(k, b)

            @pl.when(k + 2 < NCHUNK)
            def _prefetch_idx():
                issue_idx(k + 2, b)
        return carry

    lax.fori_loop(0, NCHUNK // 2, body, None)
    wait_writebacks(0)
    wait_writebacks(1)


def _gather(ha, hb, pos4, src, dst):
    mesh = plsc.VectorSubcoreMesh(core_axis_name="c", subcore_axis_name="s")
    fn = pl.kernel(
        _gather_body,
        out_type=[jax.ShapeDtypeStruct((EPAD, H), jnp.float32),
                  jax.ShapeDtypeStruct((EPAD, H), jnp.float32),
                  jax.ShapeDtypeStruct((8, EPAD), jnp.float32)],
        mesh=mesh,
        scratch_types=[pltpu.VMEM((2, CHUNK), jnp.int32),
                       pltpu.VMEM((2, CHUNK), jnp.int32),
                       pltpu.VMEM((2, CHUNK, H), jnp.float32),
                       pltpu.VMEM((2, CHUNK, H), jnp.float32),
                       pltpu.VMEM((NPAD * 4,), jnp.float32),
                       pltpu.VMEM((2, 8, CHUNK), jnp.float32),
                       pltpu.SemaphoreType.DMA,
                       pltpu.SemaphoreType.DMA,
                       pltpu.SemaphoreType.DMA,
                       pltpu.SemaphoreType.DMA,
                       pltpu.SemaphoreType.DMA,
                       pltpu.SemaphoreType.DMA],
        compiler_params=pltpu.CompilerParams(needs_layout_passes=False),
    )
    return fn(ha, hb, pos4, src, dst)


def _mlp_body(qa_ref, qb_ref, geom_ref, w1e_ref, we2_ref, be2_ref,
              wc1_ref, bc1_ref, wc2_ref, m2_ref, rw_ref):
    rel = jnp.transpose(geom_ref[...])
    dist2 = jnp.sum(rel * rel, axis=1, keepdims=True)
    m1 = _silu(qa_ref[...] + qb_ref[...] + dist2 * w1e_ref[...])
    m2 = _silu(jnp.dot(m1, we2_ref[...], preferred_element_type=jnp.float32) + be2_ref[...])
    cw = jnp.dot(_silu(jnp.dot(m2, wc1_ref[...], preferred_element_type=jnp.float32) + bc1_ref[...]),
                 wc2_ref[...], preferred_element_type=jnp.float32)
    m2_ref[...] = m2
    sub = lax.broadcasted_iota(jnp.int32, (8, EB), 0)
    rw_ref[...] = jnp.where(sub == 3, 1.0, geom_ref[...] * jnp.transpose(cw))


def _mlp(qa, qb, geom, w1e, we2, be2, wc1, bc1, wc2):
    grid = (EPAD // EB,)
    eblk = pl.BlockSpec((EB, H), lambda i: (i, 0))
    gblk = pl.BlockSpec((8, EB), lambda i: (0, i))
    full = lambda shape: pl.BlockSpec(shape, lambda i: tuple(0 for _ in shape))
    return pl.pallas_call(
        _mlp_body,
        grid=grid,
        in_specs=[eblk, eblk, gblk,
                  full((1, H)), full((H, H)), full((1, H)),
                  full((H, H)), full((1, H)), full((H, 1))],
        out_specs=[eblk, gblk],
        out_shape=[jax.ShapeDtypeStruct((EPAD, H), jnp.float32),
                   jax.ShapeDtypeStruct((8, EPAD), jnp.float32)],
    )(qa, qb, geom, w1e, we2, be2, wc1, bc1, wc2)


TPT = EPAD // NS
NCH2 = TPT // CHUNK


def _scatter_body(m2_hbm, rw_hbm, dst_hbm, z128_hbm,
                  magg_out, posacc_out,
                  acc, didx_v, m2_v, rw_tv, rw128_v, sem):
    cid = lax.axis_index("c")
    sid = lax.axis_index("s")
    rbase = sid * ROWS_PER_TILE
    pltpu.sync_copy(z128_hbm.at[pl.ds(rbase, ROWS_PER_TILE)],
                    acc.at[pl.ds(rbase, ROWS_PER_TILE)])
    plsc.subcore_barrier()

    @pl.when(cid == 1)
    def _m2_loop():
        def body(i, carry):
            base = sid * TPT + i * CHUNK
            pltpu.sync_copy(dst_hbm.at[pl.ds(base, CHUNK)], didx_v)
            pltpu.async_copy(m2_hbm.at[pl.ds(base, CHUNK)], m2_v, sem).wait()
            pltpu.sync_copy(m2_v, acc.at[didx_v], add=True)
            return carry
        lax.fori_loop(0, NCH2, body, None)

    @pl.when(cid == 0)
    def _rw_loop():
        pltpu.sync_copy(z128_hbm.at[pl.ds(0, CHUNK)], rw128_v)
        lane16 = lax.broadcasted_iota(jnp.int32, (16,), 0)

        def body(i, carry):
            base = sid * TPT + i * CHUNK
            pltpu.sync_copy(dst_hbm.at[pl.ds(base, CHUNK)], didx_v)
            pltpu.sync_copy(rw_hbm.at[:, pl.ds(base, CHUNK)], rw_tv)

            def grp(g, c2):
                e16 = g * 16 + lane16
                for c in range(4):
                    plsc.store_scatter(rw128_v, [e16, jnp.full((16,), c, jnp.int32)],
                                       rw_tv[c, pl.ds(g * 16, 16)])
                return c2
            lax.fori_loop(0, CHUNK // 16, grp, None)
            pltpu.sync_copy(rw128_v, acc.at[didx_v], add=True)
            return carry
        lax.fori_loop(0, NCH2, body, None)

    plsc.subcore_barrier()

    @pl.when(cid == 1)
    def _dump_m():
        pltpu.sync_copy(acc.at[pl.ds(rbase, ROWS_PER_TILE)],
                        magg_out.at[pl.ds(rbase, ROWS_PER_TILE)])

    @pl.when(cid == 0)
    def _dump_p():
        pltpu.sync_copy(acc.at[pl.ds(rbase, ROWS_PER_TILE)],
                        posacc_out.at[pl.ds(rbase, ROWS_PER_TILE)])


def _scatter(m2, rw, dst, z128):
    mesh = plsc.VectorSubcoreMesh(core_axis_name="c", subcore_axis_name="s")
    fn = pl.kernel(
        _scatter_body,
        out_type=[jax.ShapeDtypeStruct((NPAD, H), jnp.float32),
                  jax.ShapeDtypeStruct((NPAD, H), jnp.float32)],
        mesh=mesh,
        scratch_types=[pltpu.VMEM_SHARED((NPAD, H), jnp.float32),
                       pltpu.VMEM((CHUNK,), jnp.int32),
                       pltpu.VMEM((CHUNK, H), jnp.float32),
                       pltpu.VMEM((8, CHUNK), jnp.float32),
                       pltpu.VMEM((CHUNK, H), jnp.float32),
                       pltpu.SemaphoreType.DMA],
        compiler_params=pltpu.CompilerParams(needs_layout_passes=False),
    )
    return fn(m2, rw, dst, z128)


def _final_body(hp_ref, magg_ref, pacc_ref, pos_ref, wh1a_ref, wh1b_ref,
                bh1_ref, wh2_ref, bh2_ref, hout_ref, pout_ref):
    hp = hp_ref[...]
    magg = magg_ref[...]
    pacc = pacc_ref[...]
    t = _silu(jnp.dot(hp, wh1a_ref[...], preferred_element_type=jnp.float32)
              + jnp.dot(magg, wh1b_ref[...], preferred_element_type=jnp.float32)
              + bh1_ref[...])
    hout_ref[...] = hp + jnp.dot(t, wh2_ref[...], preferred_element_type=jnp.float32) + bh2_ref[...]
    lane = lax.broadcasted_iota(jnp.int32, (NB, H), 1)
    cnt = jnp.sum(jnp.where(lane == 3, pacc, 0.0), axis=1, keepdims=True)
    upd = jnp.where(lane < 3, pacc, 0.0) / jnp.maximum(cnt, 1.0)
    pout_ref[...] = pos_ref[...] + upd


def _final(hp, magg, pacc, pos128, wh1a, wh1b, bh1, wh2, bh2):
    grid = (NPAD // NB,)
    nblk = pl.BlockSpec((NB, D), lambda i: (i, 0))
    full = lambda shape: pl.BlockSpec(shape, lambda i: tuple(0 for _ in shape))
    return pl.pallas_call(
        _final_body,
        grid=grid,
        in_specs=[nblk, nblk, nblk, nblk,
                  full((D, H)), full((H, H)), full((1, H)),
                  full((H, D)), full((1, D))],
        out_specs=[nblk, nblk],
        out_shape=[jax.ShapeDtypeStruct((NPAD, D), jnp.float32),
                   jax.ShapeDtypeStruct((NPAD, D), jnp.float32)],
    )(hp, magg, pacc, pos128, wh1a, wh1b, bh1, wh2, bh2)


@jax.jit
def kernel(h, pos, edge_index, h_init, W_e1, b_e1, W_e2, b_e2, W_c1, b_c1,
           W_c2, W_h1, b_h1, W_h2, b_h2):
    h_pad = jnp.pad(h, ((0, NPAD - N), (0, 0)))
    hi_pad = jnp.pad(h_init, ((0, NPAD - N), (0, 0)))
    pos128 = jnp.pad(pos, ((0, NPAD - N), (0, D - 3)))
    pos4 = jnp.pad(pos, ((0, NPAD - N), (0, 1))).reshape(-1)
    src = jnp.pad(edge_index[0], (0, EPAD - E))
    dst = jnp.pad(edge_index[1], (0, EPAD - E), constant_values=N)

    w1d = W_e1[:D]
    w1s = W_e1[D:2 * D]
    w1e = W_e1[2 * D:]
    b1 = b_e1.reshape(1, H)
    be2 = b_e2.reshape(1, H)
    bc1 = b_c1.reshape(1, H)
    wh1a = W_h1[:D]
    wh1b = W_h1[D:]
    bh1 = b_h1.reshape(1, H)
    bh2 = b_h2.reshape(1, D)

    hp, ha, hb = _prep(h_pad, hi_pad, w1d, w1s, b1)
    qa, qb, geom = _gather(ha, hb, pos4, src, dst)
    m2, rw = _mlp(qa, qb, geom, w1e, W_e2, be2, W_c1, bc1, W_c2)
    z128 = jnp.zeros((NPAD, H), jnp.float32)
    magg, pacc = _scatter(m2, rw, dst, z128)
    h_out, pos_out = _final(hp, magg, pacc, pos128, wh1a, wh1b, bh1, W_h2, bh2)
    return (h_out[:N], pos_out[:N, :3])


# speedup vs baseline: 5.1303x; 1.1768x over previous
import functools

import jax
import jax.numpy as jnp
from jax import lax
from jax.experimental import pallas as pl
from jax.experimental.pallas import tpu as pltpu
from jax.experimental.pallas import tpu_sc as plsc

N = 10000
E = 320000
D = 128
H = 128

NPAD = 10240
EPAD = 327680
NC = 2
NS = 16
NW = NC * NS
EW = EPAD // NW
CHUNK = 128
NCHUNK = EW // CHUNK
ROWS_PER_TILE = NPAD // NS

NB = 1024
EB = 2048


def _silu(x):
    return x * jax.nn.sigmoid(x)


def _prep_body(h_ref, hi_ref, w1d_ref, w1s_ref, b1_ref, hp_ref, ha_ref, hb_ref):
    hp = h_ref[...] + hi_ref[...]
    hp_ref[...] = hp
    ha_ref[...] = jnp.dot(hp, w1d_ref[...], preferred_element_type=jnp.float32) + b1_ref[...]
    hb_ref[...] = jnp.dot(hp, w1s_ref[...], preferred_element_type=jnp.float32)


def _prep(h_pad, hi_pad, w1d, w1s, b1):
    grid = (NPAD // NB,)
    blk = pl.BlockSpec((NB, D), lambda i: (i, 0))
    wblk = pl.BlockSpec((D, H), lambda i: (0, 0))
    bblk = pl.BlockSpec((1, H), lambda i: (0, 0))
    return pl.pallas_call(
        _prep_body,
        grid=grid,
        in_specs=[blk, blk, wblk, wblk, bblk],
        out_specs=[blk, pl.BlockSpec((NB, H), lambda i: (i, 0)),
                   pl.BlockSpec((NB, H), lambda i: (i, 0))],
        out_shape=[jax.ShapeDtypeStruct((NPAD, D), jnp.float32),
                   jax.ShapeDtypeStruct((NPAD, H), jnp.float32),
                   jax.ShapeDtypeStruct((NPAD, H), jnp.float32)],
    )(h_pad, hi_pad, w1d, w1s, b1)


def _gather_body(ha_hbm, hb_hbm, pos_hbm, src_hbm, dst_hbm,
                 qa_out, qb_out, geom_out,
                 sidx_v, didx_v, qa_v, qb_v, pos_v, geom_v,
                 isem0, isem1, gsem0, gsem1, wsem0, wsem1):
    wid = lax.axis_index("s") * NC + lax.axis_index("c")
    isem = (isem0, isem1)
    gsem = (gsem0, gsem1)
    wsem = (wsem0, wsem1)
    pltpu.sync_copy(pos_hbm, pos_v)
    z16 = jnp.zeros((16,), jnp.float32)
    for b in range(2):
        for c in range(3, 8):
            for g in range(CHUNK // 16):
                geom_v[b, c, pl.ds(g * 16, 16)] = z16

    def issue_idx(k, b):
        base = wid * EW + k * CHUNK
        pltpu.async_copy(src_hbm.at[pl.ds(base, CHUNK)], sidx_v.at[b], isem[b])
        pltpu.async_copy(dst_hbm.at[pl.ds(base, CHUNK)], didx_v.at[b], isem[b])

    def wait_idx(b):
        pltpu.make_async_copy(src_hbm.at[pl.ds(0, CHUNK)], sidx_v.at[b], isem[b]).wait()
        pltpu.make_async_copy(dst_hbm.at[pl.ds(0, CHUNK)], didx_v.at[b], isem[b]).wait()

    def fire_gathers(b):
        pltpu.async_copy(ha_hbm.at[didx_v.at[b]], qa_v.at[b], gsem[b])
        pltpu.async_copy(hb_hbm.at[sidx_v.at[b]], qb_v.at[b], gsem[b])

    def wait_gathers(b):
        pltpu.make_async_copy(ha_hbm.at[pl.ds(0, CHUNK)], qa_v.at[b], gsem[b]).wait()
        pltpu.make_async_copy(hb_hbm.at[pl.ds(0, CHUNK)], qb_v.at[b], gsem[b]).wait()

    def compute_geom(b):
        def grp(g, carry):
            s16 = sidx_v[b, pl.ds(g * 16, 16)] * 4
            d16 = didx_v[b, pl.ds(g * 16, 16)] * 4
            for c in range(3):
                psc = plsc.load_gather(pos_v, [s16 + c])
                pdc = plsc.load_gather(pos_v, [d16 + c])
                geom_v[b, c, pl.ds(g * 16, 16)] = pdc - psc
            return carry
        lax.fori_loop(0, CHUNK // 16, grp, None)

    def fire_writebacks(k, b):
        base = wid * EW + k * CHUNK
        pltpu.async_copy(qa_v.at[b], qa_out.at[pl.ds(base, CHUNK)], wsem[b])
        pltpu.async_copy(qb_v.at[b], qb_out.at[pl.ds(base, CHUNK)], wsem[b])
        pltpu.async_copy(geom_v.at[b], geom_out.at[:, pl.ds(base, CHUNK)], wsem[b])

    def wait_writebacks(b):
        pltpu.make_async_copy(qa_v.at[b], qa_out.at[pl.ds(0, CHUNK)], wsem[b]).wait()
        pltpu.make_async_copy(qb_v.at[b], qb_out.at[pl.ds(0, CHUNK)], wsem[b]).wait()
        pltpu.make_async_copy(geom_v.at[b], geom_out.at[:, pl.ds(0, CHUNK)], wsem[b]).wait()

    issue_idx(0, 0)
    issue_idx(1, 1)
    wait_idx(0)
    fire_gathers(0)
    compute_geom(0)

    def body(j, carry):
        for b in range(2):
            k = 2 * j + b
            b1 = 1 - b

            @pl.when(k + 1 < NCHUNK)
            def _start_next():
                wait_idx(b1)

                @pl.when(k >= 1)
                def _free_slot():
                    wait_writebacks(b1)
                fire_gathers(b1)
                compute_geom(b1)

            wait_gathers(b)
            fire_writebacks(k, b)

            @pl.when(k + 2 < NCHUNK)
            def _prefetch_idx():
                issue_idx(k + 2, b)
        return carry

    lax.fori_loop(0, NCHUNK // 2, body, None)
    wait_writebacks(0)
    wait_writebacks(1)


def _gather(ha, hb, pos4, src, dst):
    mesh = plsc.VectorSubcoreMesh(core_axis_name="c", subcore_axis_name="s")
    fn = pl.kernel(
        _gather_body,
        out_type=[jax.ShapeDtypeStruct((EPAD, H), jnp.float32),
                  jax.ShapeDtypeStruct((EPAD, H), jnp.float32),
                  jax.ShapeDtypeStruct((8, EPAD), jnp.float32)],
        mesh=mesh,
        scratch_types=[pltpu.VMEM((2, CHUNK), jnp.int32),
                       pltpu.VMEM((2, CHUNK), jnp.int32),
                       pltpu.VMEM((2, CHUNK, H), jnp.float32),
                       pltpu.VMEM((2, CHUNK, H), jnp.float32),
                       pltpu.VMEM((NPAD * 4,), jnp.float32),
                       pltpu.VMEM((2, 8, CHUNK), jnp.float32),
                       pltpu.SemaphoreType.DMA,
                       pltpu.SemaphoreType.DMA,
                       pltpu.SemaphoreType.DMA,
                       pltpu.SemaphoreType.DMA,
                       pltpu.SemaphoreType.DMA,
                       pltpu.SemaphoreType.DMA],
        compiler_params=pltpu.CompilerParams(needs_layout_passes=False),
    )
    return fn(ha, hb, pos4, src, dst)


def _mlp_body(qa_ref, qb_ref, geom_ref, w1e_ref, we2_ref, be2_ref,
              wc1_ref, bc1_ref, wc2_ref, m2_ref, rw_ref):
    rel = jnp.transpose(geom_ref[...])
    dist2 = jnp.sum(rel * rel, axis=1, keepdims=True)
    m1 = _silu(qa_ref[...] + qb_ref[...] + dist2 * w1e_ref[...])
    m2 = _silu(jnp.dot(m1, we2_ref[...], preferred_element_type=jnp.float32) + be2_ref[...])
    cw = jnp.dot(_silu(jnp.dot(m2, wc1_ref[...], preferred_element_type=jnp.float32) + bc1_ref[...]),
                 wc2_ref[...], preferred_element_type=jnp.float32)
    m2_ref[...] = m2
    sub = lax.broadcasted_iota(jnp.int32, (8, EB), 0)
    rw_ref[...] = jnp.where(sub == 3, 1.0, geom_ref[...] * jnp.transpose(cw))


def _mlp(qa, qb, geom, w1e, we2, be2, wc1, bc1, wc2):
    grid = (EPAD // EB,)
    eblk = pl.BlockSpec((EB, H), lambda i: (i, 0))
    gblk = pl.BlockSpec((8, EB), lambda i: (0, i))
    full = lambda shape: pl.BlockSpec(shape, lambda i: tuple(0 for _ in shape))
    return pl.pallas_call(
        _mlp_body,
        grid=grid,
        in_specs=[eblk, eblk, gblk,
                  full((1, H)), full((H, H)), full((1, H)),
                  full((H, H)), full((1, H)), full((H, 1))],
        out_specs=[eblk, gblk],
        out_shape=[jax.ShapeDtypeStruct((EPAD, H), jnp.float32),
                   jax.ShapeDtypeStruct((8, EPAD), jnp.float32)],
    )(qa, qb, geom, w1e, we2, be2, wc1, bc1, wc2)


TPT = EPAD // NS
NCH2 = TPT // CHUNK


def _scatter_body(m2_hbm, rw_hbm, dst_hbm, z128_hbm,
                  magg_out, posacc_out,
                  acc, didx_v, m2_v, rw_tv,
                  isem0, isem1, rsem0, rsem1, asem0, asem1):
    rw128_v = m2_v
    cid = lax.axis_index("c")
    sid = lax.axis_index("s")
    rbase = sid * ROWS_PER_TILE
    isem = (isem0, isem1)
    rsem = (rsem0, rsem1)
    asem = (asem0, asem1)
    pltpu.sync_copy(z128_hbm.at[pl.ds(rbase, ROWS_PER_TILE)],
                    acc.at[pl.ds(rbase, ROWS_PER_TILE)])
    plsc.subcore_barrier()

    def issue_idx(k, b):
        base = sid * TPT + k * CHUNK
        pltpu.async_copy(dst_hbm.at[pl.ds(base, CHUNK)], didx_v.at[b], isem[b])

    def wait_idx(b):
        pltpu.make_async_copy(dst_hbm.at[pl.ds(0, CHUNK)], didx_v.at[b], isem[b]).wait()

    @pl.when(cid == 1)
    def _m2_loop():
        def issue_read(k, b):
            base = sid * TPT + k * CHUNK
            pltpu.async_copy(m2_hbm.at[pl.ds(base, CHUNK)], m2_v.at[b], rsem[b])

        issue_idx(0, 0); issue_read(0, 0)
        issue_idx(1, 1); issue_read(1, 1)

        def body(j, carry):
            for b in range(2):
                k = 2 * j + b
                wait_idx(b)
                pltpu.make_async_copy(m2_hbm.at[pl.ds(0, CHUNK)], m2_v.at[b], rsem[b]).wait()
                add = pltpu.async_copy(m2_v.at[b], acc.at[didx_v.at[b]], asem[b], add=True)
                add.wait()

                @pl.when(k + 2 < NCH2)
                def _prefetch():
                    issue_idx(k + 2, b)
                    issue_read(k + 2, b)
            return carry
        lax.fori_loop(0, NCH2 // 2, body, None)

    @pl.when(cid == 0)
    def _rw_loop():
        pltpu.sync_copy(z128_hbm.at[pl.ds(0, CHUNK)], rw128_v.at[0])
        pltpu.sync_copy(z128_hbm.at[pl.ds(0, CHUNK)], rw128_v.at[1])
        lane16 = lax.broadcasted_iota(jnp.int32, (16,), 0)

        def issue_read(k, b):
            base = sid * TPT + k * CHUNK
            pltpu.async_copy(rw_hbm.at[:, pl.ds(base, CHUNK)], rw_tv.at[b], rsem[b])

        issue_idx(0, 0); issue_read(0, 0)
        issue_idx(1, 1); issue_read(1, 1)

        def body(j, carry):
            for b in range(2):
                k = 2 * j + b
                wait_idx(b)
                pltpu.make_async_copy(rw_hbm.at[:, pl.ds(0, CHUNK)], rw_tv.at[b], rsem[b]).wait()

                def grp(g, c2):
                    e16 = g * 16 + lane16
                    for c in range(4):
                        plsc.store_scatter(rw128_v.at[b],
                                           [e16, jnp.full((16,), c, jnp.int32)],
                                           rw_tv[b, c, pl.ds(g * 16, 16)])
                    return c2
                lax.fori_loop(0, CHUNK // 16, grp, None)
                add = pltpu.async_copy(rw128_v.at[b], acc.at[didx_v.at[b]], asem[b], add=True)
                add.wait()

                @pl.when(k + 2 < NCH2)
                def _prefetch():
                    issue_idx(k + 2, b)
                    issue_read(k + 2, b)
            return carry
        lax.fori_loop(0, NCH2 // 2, body, None)

    plsc.subcore_barrier()

    @pl.when(cid == 1)
    def _dump_m():
        pltpu.sync_copy(acc.at[pl.ds(rbase, ROWS_PER_TILE)],
                        magg_out.at[pl.ds(rbase, ROWS_PER_TILE)])

    @pl.when(cid == 0)
    def _dump_p():
        pltpu.sync_copy(acc.at[pl.ds(rbase, ROWS_PER_TILE)],
                        posacc_out.at[pl.ds(rbase, ROWS_PER_TILE)])


def _scatter(m2, rw, dst, z128):
    mesh = plsc.VectorSubcoreMesh(core_axis_name="c", subcore_axis_name="s")
    fn = pl.kernel(
        _scatter_body,
        out_type=[jax.ShapeDtypeStruct((NPAD, H), jnp.float32),
                  jax.ShapeDtypeStruct((NPAD, H), jnp.float32)],
        mesh=mesh,
        scratch_types=[pltpu.VMEM_SHARED((NPAD, H), jnp.float32),
                       pltpu.VMEM((2, CHUNK), jnp.int32),
                       pltpu.VMEM((2, CHUNK, H), jnp.float32),
                       pltpu.VMEM((2, 8, CHUNK), jnp.float32),
                       pltpu.SemaphoreType.DMA,
                       pltpu.SemaphoreType.DMA,
                       pltpu.SemaphoreType.DMA,
                       pltpu.SemaphoreType.DMA,
                       pltpu.SemaphoreType.DMA,
                       pltpu.SemaphoreType.DMA],
        compiler_params=pltpu.CompilerParams(needs_layout_passes=False),
    )
    return fn(m2, rw, dst, z128)


def _final_body(hp_ref, magg_ref, pacc_ref, pos_ref, wh1a_ref, wh1b_ref,
                bh1_ref, wh2_ref, bh2_ref, hout_ref, pout_ref):
    hp = hp_ref[...]
    magg = magg_ref[...]
    pacc = pacc_ref[...]
    t = _silu(jnp.dot(hp, wh1a_ref[...], preferred_element_type=jnp.float32)
              + jnp.dot(magg, wh1b_ref[...], preferred_element_type=jnp.float32)
              + bh1_ref[...])
    hout_ref[...] = hp + jnp.dot(t, wh2_ref[...], preferred_element_type=jnp.float32) + bh2_ref[...]
    lane = lax.broadcasted_iota(jnp.int32, (NB, H), 1)
    cnt = jnp.sum(jnp.where(lane == 3, pacc, 0.0), axis=1, keepdims=True)
    upd = jnp.where(lane < 3, pacc, 0.0) / jnp.maximum(cnt, 1.0)
    pout_ref[...] = pos_ref[...] + upd


def _final(hp, magg, pacc, pos128, wh1a, wh1b, bh1, wh2, bh2):
    grid = (NPAD // NB,)
    nblk = pl.BlockSpec((NB, D), lambda i: (i, 0))
    full = lambda shape: pl.BlockSpec(shape, lambda i: tuple(0 for _ in shape))
    return pl.pallas_call(
        _final_body,
        grid=grid,
        in_specs=[nblk, nblk, nblk, nblk,
                  full((D, H)), full((H, H)), full((1, H)),
                  full((H, D)), full((1, D))],
        out_specs=[nblk, nblk],
        out_shape=[jax.ShapeDtypeStruct((NPAD, D), jnp.float32),
                   jax.ShapeDtypeStruct((NPAD, D), jnp.float32)],
    )(hp, magg, pacc, pos128, wh1a, wh1b, bh1, wh2, bh2)


@jax.jit
def kernel(h, pos, edge_index, h_init, W_e1, b_e1, W_e2, b_e2, W_c1, b_c1,
           W_c2, W_h1, b_h1, W_h2, b_h2):
    h_pad = jnp.pad(h, ((0, NPAD - N), (0, 0)))
    hi_pad = jnp.pad(h_init, ((0, NPAD - N), (0, 0)))
    pos128 = jnp.pad(pos, ((0, NPAD - N), (0, D - 3)))
    pos4 = jnp.pad(pos, ((0, NPAD - N), (0, 1))).reshape(-1)
    src = jnp.pad(edge_index[0], (0, EPAD - E))
    dst = jnp.pad(edge_index[1], (0, EPAD - E), constant_values=N)

    w1d = W_e1[:D]
    w1s = W_e1[D:2 * D]
    w1e = W_e1[2 * D:]
    b1 = b_e1.reshape(1, H)
    be2 = b_e2.reshape(1, H)
    bc1 = b_c1.reshape(1, H)
    wh1a = W_h1[:D]
    wh1b = W_h1[D:]
    bh1 = b_h1.reshape(1, H)
    bh2 = b_h2.reshape(1, D)

    hp, ha, hb = _prep(h_pad, hi_pad, w1d, w1s, b1)
    qa, qb, geom = _gather(ha, hb, pos4, src, dst)
    m2, rw = _mlp(qa, qb, geom, w1e, W_e2, be2, W_c1, bc1, W_c2)
    z128 = jnp.zeros((NPAD, H), jnp.float32)
    magg, pacc = _scatter(m2, rw, dst, z128)
    h_out, pos_out = _final(hp, magg, pacc, pos128, wh1a, wh1b, bh1, W_h2, bh2)
    return (h_out[:N], pos_out[:N, :3])


# speedup vs baseline: 5.1410x; 1.0021x over previous
import functools

import jax
import jax.numpy as jnp
from jax import lax
from jax.experimental import pallas as pl
from jax.experimental.pallas import tpu as pltpu
from jax.experimental.pallas import tpu_sc as plsc

N = 10000
E = 320000
D = 128
H = 128

NPAD = 10240
EPAD = 327680
NC = 2
NS = 16
NW = NC * NS
EW = EPAD // NW
CHUNK = 128
NCHUNK = EW // CHUNK
ROWS_PER_TILE = NPAD // NS

NB = 1024
EB = 2048


def _silu(x):
    return x * jax.nn.sigmoid(x)


def _prep_body(h_ref, hi_ref, w1d_ref, w1s_ref, b1_ref, hp_ref, ha_ref, hb_ref):
    hp = h_ref[...] + hi_ref[...]
    hp_ref[...] = hp
    ha_ref[...] = jnp.dot(hp, w1d_ref[...], preferred_element_type=jnp.float32) + b1_ref[...]
    hb_ref[...] = jnp.dot(hp, w1s_ref[...], preferred_element_type=jnp.float32)


def _prep(h_pad, hi_pad, w1d, w1s, b1):
    grid = (NPAD // NB,)
    blk = pl.BlockSpec((NB, D), lambda i: (i, 0))
    wblk = pl.BlockSpec((D, H), lambda i: (0, 0))
    bblk = pl.BlockSpec((1, H), lambda i: (0, 0))
    return pl.pallas_call(
        _prep_body,
        grid=grid,
        in_specs=[blk, blk, wblk, wblk, bblk],
        out_specs=[blk, pl.BlockSpec((NB, H), lambda i: (i, 0)),
                   pl.BlockSpec((NB, H), lambda i: (i, 0))],
        out_shape=[jax.ShapeDtypeStruct((NPAD, D), jnp.float32),
                   jax.ShapeDtypeStruct((NPAD, H), jnp.float32),
                   jax.ShapeDtypeStruct((NPAD, H), jnp.float32)],
    )(h_pad, hi_pad, w1d, w1s, b1)


CG = 64
NSLOT = 4
NCHG = EW // CG


def _gather_body(ha_hbm, hb_hbm, pos_hbm, src_hbm, dst_hbm,
                 qa_out, qb_out, geom_out,
                 sidx_v, didx_v, qa_v, qb_v, pos_v, geom_v, *sems):
    wid = lax.axis_index("s") * NC + lax.axis_index("c")
    isem = sems[0:NSLOT]
    gsem = sems[NSLOT:2 * NSLOT]
    wsem = sems[2 * NSLOT:3 * NSLOT]
    gwsem = sems[3 * NSLOT:3 * NSLOT + 2]
    pltpu.sync_copy(pos_hbm, pos_v)
    z16 = jnp.zeros((16,), jnp.float32)
    for p in range(2):
        for c in range(3, 8):
            for g in range(128 // 16):
                geom_v[p, c, pl.ds(g * 16, 16)] = z16

    def issue_idx(k, b):
        base = wid * EW + k * CG
        pltpu.async_copy(src_hbm.at[pl.ds(base, CG)], sidx_v.at[b], isem[b])
        pltpu.async_copy(dst_hbm.at[pl.ds(base, CG)], didx_v.at[b], isem[b])

    def wait_idx(b):
        pltpu.make_async_copy(src_hbm.at[pl.ds(0, CG)], sidx_v.at[b], isem[b]).wait()
        pltpu.make_async_copy(dst_hbm.at[pl.ds(0, CG)], didx_v.at[b], isem[b]).wait()

    def fire_gathers(b):
        pltpu.async_copy(ha_hbm.at[didx_v.at[b]], qa_v.at[b], gsem[b])
        pltpu.async_copy(hb_hbm.at[sidx_v.at[b]], qb_v.at[b], gsem[b])

    def wait_gathers(b):
        pltpu.make_async_copy(ha_hbm.at[pl.ds(0, CG)], qa_v.at[b], gsem[b]).wait()
        pltpu.make_async_copy(hb_hbm.at[pl.ds(0, CG)], qb_v.at[b], gsem[b]).wait()

    def compute_geom(b, p, half):
        off = half * CG
        def grp(g, carry):
            s16 = sidx_v[b, pl.ds(g * 16, 16)] * 4
            d16 = didx_v[b, pl.ds(g * 16, 16)] * 4
            for c in range(3):
                psc = plsc.load_gather(pos_v, [s16 + c])
                pdc = plsc.load_gather(pos_v, [d16 + c])
                geom_v[p, c, pl.ds(off + g * 16, 16)] = pdc - psc
            return carry
        lax.fori_loop(0, CG // 16, grp, None)

    def fire_writebacks(k, b):
        base = wid * EW + k * CG
        pltpu.async_copy(qa_v.at[b], qa_out.at[pl.ds(base, CG)], wsem[b])
        pltpu.async_copy(qb_v.at[b], qb_out.at[pl.ds(base, CG)], wsem[b])

    def wait_writebacks(b):
        pltpu.make_async_copy(qa_v.at[b], qa_out.at[pl.ds(0, CG)], wsem[b]).wait()
        pltpu.make_async_copy(qb_v.at[b], qb_out.at[pl.ds(0, CG)], wsem[b]).wait()

    def fire_geom_wb(k, p):
        base = wid * EW + (k - 1) * CG
        pltpu.async_copy(geom_v.at[p], geom_out.at[:, pl.ds(base, 2 * CG)], gwsem[p])

    def wait_geom_wb(p):
        pltpu.make_async_copy(geom_v.at[p], geom_out.at[:, pl.ds(0, 2 * CG)], gwsem[p]).wait()

    for b in range(NSLOT):
        issue_idx(b, b)

    def body(j, carry):
        for kk in range(NSLOT):
            k = NSLOT * j + kk
            b = kk
            b2 = (kk + 1) % NSLOT

            wait_idx(b)

            @pl.when(k >= NSLOT)
            def _free_slot():
                wait_writebacks(b)
            fire_gathers(b)
            p = (kk // 2) % 2
            if kk % 2 == 0:
                @pl.when(k >= NSLOT)
                def _free_geom_pair():
                    wait_geom_wb(p)
            compute_geom(b, p, kk % 2)
            if kk % 2 == 1:
                fire_geom_wb(k, p)

            @pl.when(k >= NSLOT - 1)
            def _finish_old():
                jj = k - (NSLOT - 1)
                wait_gathers(b2)
                fire_writebacks(jj, b2)

                @pl.when(jj + NSLOT < NCHG)
                def _prefetch_idx():
                    issue_idx(jj + NSLOT, b2)
        return carry

    lax.fori_loop(0, NCHG // NSLOT, body, None)
    for t in range(NCHG - (NSLOT - 1), NCHG):
        b2 = t % NSLOT
        wait_gathers(b2)
        fire_writebacks(t, b2)
    for b in range(NSLOT):
        wait_writebacks(b)
    for p in range(2):
        wait_geom_wb(p)


def _gather(ha, hb, pos4, src, dst):
    mesh = plsc.VectorSubcoreMesh(core_axis_name="c", subcore_axis_name="s")
    fn = pl.kernel(
        _gather_body,
        out_type=[jax.ShapeDtypeStruct((EPAD, H), jnp.float32),
                  jax.ShapeDtypeStruct((EPAD, H), jnp.float32),
                  jax.ShapeDtypeStruct((8, EPAD), jnp.float32)],
        mesh=mesh,
        scratch_types=[pltpu.VMEM((NSLOT, CG), jnp.int32),
                       pltpu.VMEM((NSLOT, CG), jnp.int32),
                       pltpu.VMEM((NSLOT, CG, H), jnp.float32),
                       pltpu.VMEM((NSLOT, CG, H), jnp.float32),
                       pltpu.VMEM((NPAD * 4,), jnp.float32),
                       pltpu.VMEM((2, 8, 2 * CG), jnp.float32)]
                      + [pltpu.SemaphoreType.DMA] * (3 * NSLOT + 2),
        compiler_params=pltpu.CompilerParams(needs_layout_passes=False),
    )
    return fn(ha, hb, pos4, src, dst)


def _mlp_body(qa_ref, qb_ref, geom_ref, w1e_ref, we2_ref, be2_ref,
              wc1_ref, bc1_ref, wc2_ref, m2_ref, rw_ref):
    rel = jnp.transpose(geom_ref[...])
    dist2 = jnp.sum(rel * rel, axis=1, keepdims=True)
    m1 = _silu(qa_ref[...] + qb_ref[...] + dist2 * w1e_ref[...])
    m2 = _silu(jnp.dot(m1, we2_ref[...], preferred_element_type=jnp.float32) + be2_ref[...])
    cw = jnp.dot(_silu(jnp.dot(m2, wc1_ref[...], preferred_element_type=jnp.float32) + bc1_ref[...]),
                 wc2_ref[...], preferred_element_type=jnp.float32)
    m2_ref[...] = m2
    sub = lax.broadcasted_iota(jnp.int32, (8, EB), 0)
    rw_ref[...] = jnp.where(sub == 3, 1.0, geom_ref[...] * jnp.transpose(cw))


def _mlp(qa, qb, geom, w1e, we2, be2, wc1, bc1, wc2):
    grid = (EPAD // EB,)
    eblk = pl.BlockSpec((EB, H), lambda i: (i, 0))
    gblk = pl.BlockSpec((8, EB), lambda i: (0, i))
    full = lambda shape: pl.BlockSpec(shape, lambda i: tuple(0 for _ in shape))
    return pl.pallas_call(
        _mlp_body,
        grid=grid,
        in_specs=[eblk, eblk, gblk,
                  full((1, H)), full((H, H)), full((1, H)),
                  full((H, H)), full((1, H)), full((H, 1))],
        out_specs=[eblk, gblk],
        out_shape=[jax.ShapeDtypeStruct((EPAD, H), jnp.float32),
                   jax.ShapeDtypeStruct((8, EPAD), jnp.float32)],
    )(qa, qb, geom, w1e, we2, be2, wc1, bc1, wc2)


TPT = EPAD // NS
NCH2 = TPT // CHUNK


def _scatter_body(m2_hbm, rw_hbm, dst_hbm, z128_hbm,
                  magg_out, posacc_out,
                  acc, didx_v, m2_v, rw_tv,
                  isem0, isem1, rsem0, rsem1, asem0, asem1):
    rw128_v = m2_v
    cid = lax.axis_index("c")
    sid = lax.axis_index("s")
    rbase = sid * ROWS_PER_TILE
    isem = (isem0, isem1)
    rsem = (rsem0, rsem1)
    asem = (asem0, asem1)
    pltpu.sync_copy(z128_hbm.at[pl.ds(rbase, ROWS_PER_TILE)],
                    acc.at[pl.ds(rbase, ROWS_PER_TILE)])
    plsc.subcore_barrier()

    def issue_idx(k, b):
        base = sid * TPT + k * CHUNK
        pltpu.async_copy(dst_hbm.at[pl.ds(base, CHUNK)], didx_v.at[b], isem[b])

    def wait_idx(b):
        pltpu.make_async_copy(dst_hbm.at[pl.ds(0, CHUNK)], didx_v.at[b], isem[b]).wait()

    @pl.when(cid == 1)
    def _m2_loop():
        def issue_read(k, b):
            base = sid * TPT + k * CHUNK
            pltpu.async_copy(m2_hbm.at[pl.ds(base, CHUNK)], m2_v.at[b], rsem[b])

        issue_idx(0, 0); issue_read(0, 0)
        issue_idx(1, 1); issue_read(1, 1)

        def body(j, carry):
            for b in range(2):
                k = 2 * j + b
                wait_idx(b)
                pltpu.make_async_copy(m2_hbm.at[pl.ds(0, CHUNK)], m2_v.at[b], rsem[b]).wait()
                add = pltpu.async_copy(m2_v.at[b], acc.at[didx_v.at[b]], asem[b], add=True)
                add.wait()

                @pl.when(k + 2 < NCH2)
                def _prefetch():
                    issue_idx(k + 2, b)
                    issue_read(k + 2, b)
            return carry
        lax.fori_loop(0, NCH2 // 2, body, None)

    @pl.when(cid == 0)
    def _rw_loop():
        pltpu.sync_copy(z128_hbm.at[pl.ds(0, CHUNK)], rw128_v.at[0])
        pltpu.sync_copy(z128_hbm.at[pl.ds(0, CHUNK)], rw128_v.at[1])
        lane16 = lax.broadcasted_iota(jnp.int32, (16,), 0)

        def issue_read(k, b):
            base = sid * TPT + k * CHUNK
            pltpu.async_copy(rw_hbm.at[:, pl.ds(base, CHUNK)], rw_tv.at[b], rsem[b])

        issue_idx(0, 0); issue_read(0, 0)
        issue_idx(1, 1); issue_read(1, 1)

        def body(j, carry):
            for b in range(2):
                k = 2 * j + b
                wait_idx(b)
                pltpu.make_async_copy(rw_hbm.at[:, pl.ds(0, CHUNK)], rw_tv.at[b], rsem[b]).wait()

                def grp(g, c2):
                    e16 = g * 16 + lane16
                    for c in range(4):
                        plsc.store_scatter(rw128_v.at[b],
                                           [e16, jnp.full((16,), c, jnp.int32)],
                                           rw_tv[b, c, pl.ds(g * 16, 16)])
                    return c2
                lax.fori_loop(0, CHUNK // 16, grp, None)
                add = pltpu.async_copy(rw128_v.at[b], acc.at[didx_v.at[b]], asem[b], add=True)
                add.wait()

                @pl.when(k + 2 < NCH2)
                def _prefetch():
                    issue_idx(k + 2, b)
                    issue_read(k + 2, b)
            return carry
        lax.fori_loop(0, NCH2 // 2, body, None)

    plsc.subcore_barrier()

    @pl.when(cid == 1)
    def _dump_m():
        pltpu.sync_copy(acc.at[pl.ds(rbase, ROWS_PER_TILE)],
                        magg_out.at[pl.ds(rbase, ROWS_PER_TILE)])

    @pl.when(cid == 0)
    def _dump_p():
        pltpu.sync_copy(acc.at[pl.ds(rbase, ROWS_PER_TILE)],
                        posacc_out.at[pl.ds(rbase, ROWS_PER_TILE)])


def _scatter(m2, rw, dst, z128):
    mesh = plsc.VectorSubcoreMesh(core_axis_name="c", subcore_axis_name="s")
    fn = pl.kernel(
        _scatter_body,
        out_type=[jax.ShapeDtypeStruct((NPAD, H), jnp.float32),
                  jax.ShapeDtypeStruct((NPAD, H), jnp.float32)],
        mesh=mesh,
        scratch_types=[pltpu.VMEM_SHARED((NPAD, H), jnp.float32),
                       pltpu.VMEM((2, CHUNK), jnp.int32),
                       pltpu.VMEM((2, CHUNK, H), jnp.float32),
                       pltpu.VMEM((2, 8, CHUNK), jnp.float32),
                       pltpu.SemaphoreType.DMA,
                       pltpu.SemaphoreType.DMA,
                       pltpu.SemaphoreType.DMA,
                       pltpu.SemaphoreType.DMA,
                       pltpu.SemaphoreType.DMA,
                       pltpu.SemaphoreType.DMA],
        compiler_params=pltpu.CompilerParams(needs_layout_passes=False),
    )
    return fn(m2, rw, dst, z128)


def _final_body(hp_ref, magg_ref, pacc_ref, pos_ref, wh1a_ref, wh1b_ref,
                bh1_ref, wh2_ref, bh2_ref, hout_ref, pout_ref):
    hp = hp_ref[...]
    magg = magg_ref[...]
    pacc = pacc_ref[...]
    t = _silu(jnp.dot(hp, wh1a_ref[...], preferred_element_type=jnp.float32)
              + jnp.dot(magg, wh1b_ref[...], preferred_element_type=jnp.float32)
              + bh1_ref[...])
    hout_ref[...] = hp + jnp.dot(t, wh2_ref[...], preferred_element_type=jnp.float32) + bh2_ref[...]
    lane = lax.broadcasted_iota(jnp.int32, (NB, H), 1)
    cnt = jnp.sum(jnp.where(lane == 3, pacc, 0.0), axis=1, keepdims=True)
    upd = jnp.where(lane < 3, pacc, 0.0) / jnp.maximum(cnt, 1.0)
    pout_ref[...] = pos_ref[...] + upd


def _final(hp, magg, pacc, pos128, wh1a, wh1b, bh1, wh2, bh2):
    grid = (NPAD // NB,)
    nblk = pl.BlockSpec((NB, D), lambda i: (i, 0))
    full = lambda shape: pl.BlockSpec(shape, lambda i: tuple(0 for _ in shape))
    return pl.pallas_call(
        _final_body,
        grid=grid,
        in_specs=[nblk, nblk, nblk, nblk,
                  full((D, H)), full((H, H)), full((1, H)),
                  full((H, D)), full((1, D))],
        out_specs=[nblk, nblk],
        out_shape=[jax.ShapeDtypeStruct((NPAD, D), jnp.float32),
                   jax.ShapeDtypeStruct((NPAD, D), jnp.float32)],
    )(hp, magg, pacc, pos128, wh1a, wh1b, bh1, wh2, bh2)


@jax.jit
def kernel(h, pos, edge_index, h_init, W_e1, b_e1, W_e2, b_e2, W_c1, b_c1,
           W_c2, W_h1, b_h1, W_h2, b_h2):
    h_pad = jnp.pad(h, ((0, NPAD - N), (0, 0)))
    hi_pad = jnp.pad(h_init, ((0, NPAD - N), (0, 0)))
    pos128 = jnp.pad(pos, ((0, NPAD - N), (0, D - 3)))
    pos4 = jnp.pad(pos, ((0, NPAD - N), (0, 1))).reshape(-1)
    src = jnp.pad(edge_index[0], (0, EPAD - E))
    dst = jnp.pad(edge_index[1], (0, EPAD - E), constant_values=N)

    w1d = W_e1[:D]
    w1s = W_e1[D:2 * D]
    w1e = W_e1[2 * D:]
    b1 = b_e1.reshape(1, H)
    be2 = b_e2.reshape(1, H)
    bc1 = b_c1.reshape(1, H)
    wh1a = W_h1[:D]
    wh1b = W_h1[D:]
    bh1 = b_h1.reshape(1, H)
    bh2 = b_h2.reshape(1, D)

    hp, ha, hb = _prep(h_pad, hi_pad, w1d, w1s, b1)
    qa, qb, geom = _gather(ha, hb, pos4, src, dst)
    m2, rw = _mlp(qa, qb, geom, w1e, W_e2, be2, W_c1, bc1, W_c2)
    z128 = jnp.zeros((NPAD, H), jnp.float32)
    magg, pacc = _scatter(m2, rw, dst, z128)
    h_out, pos_out = _final(hp, magg, pacc, pos128, wh1a, wh1b, bh1, W_h2, bh2)
    return (h_out[:N], pos_out[:N, :3])


# speedup vs baseline: 8.3833x; 1.6307x over previous
import functools

import jax
import jax.numpy as jnp
from jax import lax
from jax.experimental import pallas as pl
from jax.experimental.pallas import tpu as pltpu
from jax.experimental.pallas import tpu_sc as plsc

N = 10000
E = 320000
D = 128
H = 128

NPAD = 10240
EPAD = 327680
NC = 2
NS = 16
NW = NC * NS
EW = EPAD // NW
CHUNK = 128
NCHUNK = EW // CHUNK
ROWS_PER_TILE = NPAD // NS

NB = 1024
EB = 2048


def _silu(x):
    return x * jax.nn.sigmoid(x)


def _prep_body(h_ref, hi_ref, w1d_ref, w1s_ref, b1_ref, hp_ref, ha_ref, hb_ref):
    hp = h_ref[...] + hi_ref[...]
    hp_ref[...] = hp
    ha_ref[...] = jnp.dot(hp, w1d_ref[...], preferred_element_type=jnp.float32) + b1_ref[...]
    hb_ref[...] = jnp.dot(hp, w1s_ref[...], preferred_element_type=jnp.float32)


def _prep(h_pad, hi_pad, w1d, w1s, b1):
    grid = (NPAD // NB,)
    blk = pl.BlockSpec((NB, D), lambda i: (i, 0))
    wblk = pl.BlockSpec((D, H), lambda i: (0, 0))
    bblk = pl.BlockSpec((1, H), lambda i: (0, 0))
    return pl.pallas_call(
        _prep_body,
        grid=grid,
        in_specs=[blk, blk, wblk, wblk, bblk],
        out_specs=[blk, pl.BlockSpec((NB, H), lambda i: (i, 0)),
                   pl.BlockSpec((NB, H), lambda i: (i, 0))],
        out_shape=[jax.ShapeDtypeStruct((NPAD, D), jnp.float32),
                   jax.ShapeDtypeStruct((NPAD, H), jnp.float32),
                   jax.ShapeDtypeStruct((NPAD, H), jnp.float32)],
    )(h_pad, hi_pad, w1d, w1s, b1)


CG = 64
NSLOT = 2
UNROLL = 4
TPTG = EPAD // NS
NCHG = TPTG // CG


def _gather_body(ha_hbm, hb_hbm, pos_hbm, src_hbm, dst_hbm,
                 qa_out, qb_out, geom_out,
                 table_sp, sidx_v, didx_v, q_v, pos_v, geom_v, *sems):
    cid = lax.axis_index("c")
    sid = lax.axis_index("s")
    isem = sems[0:NSLOT]
    gsem = sems[NSLOT:2 * NSLOT]
    wsem = sems[2 * NSLOT:3 * NSLOT]
    gwsem = sems[3 * NSLOT:3 * NSLOT + 2]

    rbase = sid * ROWS_PER_TILE

    @pl.when(cid == 0)
    def _stage_a():
        pltpu.sync_copy(ha_hbm.at[pl.ds(rbase, ROWS_PER_TILE)],
                        table_sp.at[pl.ds(rbase, ROWS_PER_TILE)])

    @pl.when(cid == 1)
    def _stage_b():
        pltpu.sync_copy(hb_hbm.at[pl.ds(rbase, ROWS_PER_TILE)],
                        table_sp.at[pl.ds(rbase, ROWS_PER_TILE)])

    @pl.when(cid == 0)
    def _stage_pos():
        pltpu.sync_copy(pos_hbm, pos_v)
        z16 = jnp.zeros((16,), jnp.float32)
        for p in range(2):
            for c in range(3, 8):
                for g in range(128 // 16):
                    geom_v[p, c, pl.ds(g * 16, 16)] = z16

    plsc.subcore_barrier()

    def issue_idx(k, b):
        base = sid * TPTG + k * CG

        @pl.when(cid == 0)
        def _i0():
            pltpu.async_copy(src_hbm.at[pl.ds(base, CG)], sidx_v.at[b], isem[b])
            pltpu.async_copy(dst_hbm.at[pl.ds(base, CG)], didx_v.at[b], isem[b])

        @pl.when(cid == 1)
        def _i1():
            pltpu.async_copy(src_hbm.at[pl.ds(base, CG)], sidx_v.at[b], isem[b])

    def wait_idx(b):
        pltpu.make_async_copy(src_hbm.at[pl.ds(0, CG)], sidx_v.at[b], isem[b]).wait()

        @pl.when(cid == 0)
        def _w0():
            pltpu.make_async_copy(dst_hbm.at[pl.ds(0, CG)], didx_v.at[b], isem[b]).wait()

    def fire_gathers(b):
        @pl.when(cid == 0)
        def _g0():
            pltpu.async_copy(table_sp.at[didx_v.at[b]], q_v.at[b], gsem[b])

        @pl.when(cid == 1)
        def _g1():
            pltpu.async_copy(table_sp.at[sidx_v.at[b]], q_v.at[b], gsem[b])

    def wait_gathers(b):
        pltpu.make_async_copy(table_sp.at[pl.ds(0, CG)], q_v.at[b], gsem[b]).wait()

    M_HI = jnp.int32(-65536)

    def compute_geom(b, p, half):
        off = half * CG

        def grp(g, carry):
            s2 = sidx_v[b, pl.ds(g * 16, 16)] * 2
            d2 = didx_v[b, pl.ds(g * 16, 16)] * 2
            ws = plsc.bitcast(plsc.load_gather(pos_v, [s2]), jnp.int32)
            wd = plsc.bitcast(plsc.load_gather(pos_v, [d2]), jnp.int32)
            xs = plsc.bitcast(ws & M_HI, jnp.float32)
            xd = plsc.bitcast(wd & M_HI, jnp.float32)
            ys = plsc.bitcast(lax.shift_left(ws, 16), jnp.float32)
            yd = plsc.bitcast(lax.shift_left(wd, 16), jnp.float32)
            zs = plsc.load_gather(pos_v, [s2 + 1])
            zd = plsc.load_gather(pos_v, [d2 + 1])
            geom_v[p, 0, pl.ds(off + g * 16, 16)] = xd - xs
            geom_v[p, 1, pl.ds(off + g * 16, 16)] = yd - ys
            geom_v[p, 2, pl.ds(off + g * 16, 16)] = zd - zs
            return carry
        lax.fori_loop(0, CG // 16, grp, None)

    def fire_writebacks(k, b):
        base = sid * TPTG + k * CG

        @pl.when(cid == 0)
        def _w0():
            pltpu.async_copy(q_v.at[b], qa_out.at[pl.ds(base, CG)], wsem[b])

        @pl.when(cid == 1)
        def _w1():
            pltpu.async_copy(q_v.at[b], qb_out.at[pl.ds(base, CG)], wsem[b])

    def wait_writebacks(b):
        pltpu.make_async_copy(q_v.at[b], qa_out.at[pl.ds(0, CG)], wsem[b]).wait()

    def fire_geom_wb(k, p):
        base = sid * TPTG + (k - 1) * CG
        pltpu.async_copy(geom_v.at[p], geom_out.at[:, pl.ds(base, 2 * CG)], gwsem[p])

    def wait_geom_wb(p):
        pltpu.make_async_copy(geom_v.at[p], geom_out.at[:, pl.ds(0, 2 * CG)], gwsem[p]).wait()

    for b in range(NSLOT):
        issue_idx(b, b)

    def body(j, carry):
        for kk in range(UNROLL):
            k = UNROLL * j + kk
            b = kk % NSLOT
            b2 = (kk + 1) % NSLOT
            p = (kk // 2) % 2

            wait_idx(b)

            @pl.when(k >= NSLOT)
            def _free_slot():
                wait_writebacks(b)
            fire_gathers(b)

            @pl.when(cid == 0)
            def _geom():
                if kk % 2 == 0:
                    @pl.when(k >= 4)
                    def _free_geom_pair():
                        wait_geom_wb(p)
                compute_geom(b, p, kk % 2)
                if kk % 2 == 1:
                    fire_geom_wb(k, p)

            @pl.when(k >= NSLOT - 1)
            def _finish_old():
                jj = k - (NSLOT - 1)
                wait_gathers(b2)
                fire_writebacks(jj, b2)

                @pl.when(jj + NSLOT < NCHG)
                def _prefetch_idx():
                    issue_idx(jj + NSLOT, b2)
        return carry

    lax.fori_loop(0, NCHG // UNROLL, body, None)
    for t in range(NCHG - (NSLOT - 1), NCHG):
        b2 = t % NSLOT
        wait_gathers(b2)
        fire_writebacks(t, b2)
    for b in range(NSLOT):
        wait_writebacks(b)

    @pl.when(cid == 0)
    def _drain_geom():
        for p in range(2):
            wait_geom_wb(p)


def _gather(ha, hb, pos2, src, dst):
    mesh = plsc.VectorSubcoreMesh(core_axis_name="c", subcore_axis_name="s")
    fn = pl.kernel(
        _gather_body,
        out_type=[jax.ShapeDtypeStruct((EPAD, H), jnp.float32),
                  jax.ShapeDtypeStruct((EPAD, H), jnp.float32),
                  jax.ShapeDtypeStruct((8, EPAD), jnp.float32)],
        mesh=mesh,
        scratch_types=[pltpu.VMEM_SHARED((NPAD, H), jnp.float32),
                       pltpu.VMEM((NSLOT, CG), jnp.int32),
                       pltpu.VMEM((NSLOT, CG), jnp.int32),
                       pltpu.VMEM((NSLOT, CG, H), jnp.float32),
                       pltpu.VMEM((NPAD * 2,), jnp.float32),
                       pltpu.VMEM((2, 8, 2 * CG), jnp.float32)]
                      + [pltpu.SemaphoreType.DMA] * (3 * NSLOT + 2),
        compiler_params=pltpu.CompilerParams(needs_layout_passes=False),
    )
    return fn(ha, hb, pos2, src, dst)


def _mlp_body(qa_ref, qb_ref, geom_ref, w1e_ref, we2_ref, be2_ref,
              wc1_ref, bc1_ref, wc2_ref, m2_ref, rw_ref):
    rel = jnp.transpose(geom_ref[...])
    dist2 = jnp.sum(rel * rel, axis=1, keepdims=True)
    m1 = _silu(qa_ref[...] + qb_ref[...] + dist2 * w1e_ref[...])
    m2 = _silu(jnp.dot(m1, we2_ref[...], preferred_element_type=jnp.float32) + be2_ref[...])
    cw = jnp.dot(_silu(jnp.dot(m2, wc1_ref[...], preferred_element_type=jnp.float32) + bc1_ref[...]),
                 wc2_ref[...], preferred_element_type=jnp.float32)
    m2_ref[...] = m2
    sub = lax.broadcasted_iota(jnp.int32, (8, EB), 0)
    rw_ref[...] = jnp.where(sub == 3, 1.0, geom_ref[...] * jnp.transpose(cw))


def _mlp(qa, qb, geom, w1e, we2, be2, wc1, bc1, wc2):
    grid = (EPAD // EB,)
    eblk = pl.BlockSpec((EB, H), lambda i: (i, 0))
    gblk = pl.BlockSpec((8, EB), lambda i: (0, i))
    full = lambda shape: pl.BlockSpec(shape, lambda i: tuple(0 for _ in shape))
    return pl.pallas_call(
        _mlp_body,
        grid=grid,
        in_specs=[eblk, eblk, gblk,
                  full((1, H)), full((H, H)), full((1, H)),
                  full((H, H)), full((1, H)), full((H, 1))],
        out_specs=[eblk, gblk],
        out_shape=[jax.ShapeDtypeStruct((EPAD, H), jnp.float32),
                   jax.ShapeDtypeStruct((8, EPAD), jnp.float32)],
    )(qa, qb, geom, w1e, we2, be2, wc1, bc1, wc2)


TPT = EPAD // NS
NCH2 = TPT // CHUNK


def _scatter_body(m2_hbm, rw_hbm, dst_hbm, z128_hbm,
                  magg_out, posacc_out,
                  acc, didx_v, m2_v, rw_tv,
                  isem0, isem1, rsem0, rsem1, asem0, asem1):
    rw128_v = m2_v
    cid = lax.axis_index("c")
    sid = lax.axis_index("s")
    rbase = sid * ROWS_PER_TILE
    isem = (isem0, isem1)
    rsem = (rsem0, rsem1)
    asem = (asem0, asem1)
    pltpu.sync_copy(z128_hbm.at[pl.ds(rbase, ROWS_PER_TILE)],
                    acc.at[pl.ds(rbase, ROWS_PER_TILE)])
    plsc.subcore_barrier()

    def issue_idx(k, b):
        base = sid * TPT + k * CHUNK
        pltpu.async_copy(dst_hbm.at[pl.ds(base, CHUNK)], didx_v.at[b], isem[b])

    def wait_idx(b):
        pltpu.make_async_copy(dst_hbm.at[pl.ds(0, CHUNK)], didx_v.at[b], isem[b]).wait()

    @pl.when(cid == 1)
    def _m2_loop():
        def issue_read(k, b):
            base = sid * TPT + k * CHUNK
            pltpu.async_copy(m2_hbm.at[pl.ds(base, CHUNK)], m2_v.at[b], rsem[b])

        issue_idx(0, 0); issue_read(0, 0)
        issue_idx(1, 1); issue_read(1, 1)

        def body(j, carry):
            for b in range(2):
                k = 2 * j + b
                wait_idx(b)
                pltpu.make_async_copy(m2_hbm.at[pl.ds(0, CHUNK)], m2_v.at[b], rsem[b]).wait()
                add = pltpu.async_copy(m2_v.at[b], acc.at[didx_v.at[b]], asem[b], add=True)
                add.wait()

                @pl.when(k + 2 < NCH2)
                def _prefetch():
                    issue_idx(k + 2, b)
                    issue_read(k + 2, b)
            return carry
        lax.fori_loop(0, NCH2 // 2, body, None)

    @pl.when(cid == 0)
    def _rw_loop():
        pltpu.sync_copy(z128_hbm.at[pl.ds(0, CHUNK)], rw128_v.at[0])
        pltpu.sync_copy(z128_hbm.at[pl.ds(0, CHUNK)], rw128_v.at[1])
        lane16 = lax.broadcasted_iota(jnp.int32, (16,), 0)

        def issue_read(k, b):
            base = sid * TPT + k * CHUNK
            pltpu.async_copy(rw_hbm.at[:, pl.ds(base, CHUNK)], rw_tv.at[b], rsem[b])

        issue_idx(0, 0); issue_read(0, 0)
        issue_idx(1, 1); issue_read(1, 1)

        def body(j, carry):
            for b in range(2):
                k = 2 * j + b
                wait_idx(b)
                pltpu.make_async_copy(rw_hbm.at[:, pl.ds(0, CHUNK)], rw_tv.at[b], rsem[b]).wait()

                def grp(g, c2):
                    e16 = g * 16 + lane16
                    for c in range(4):
                        plsc.store_scatter(rw128_v.at[b],
                                           [e16, jnp.full((16,), c, jnp.int32)],
                                           rw_tv[b, c, pl.ds(g * 16, 16)])
                    return c2
                lax.fori_loop(0, CHUNK // 16, grp, None)
                add = pltpu.async_copy(rw128_v.at[b], acc.at[didx_v.at[b]], asem[b], add=True)
                add.wait()

                @pl.when(k + 2 < NCH2)
                def _prefetch():
                    issue_idx(k + 2, b)
                    issue_read(k + 2, b)
            return carry
        lax.fori_loop(0, NCH2 // 2, body, None)

    plsc.subcore_barrier()

    @pl.when(cid == 1)
    def _dump_m():
        pltpu.sync_copy(acc.at[pl.ds(rbase, ROWS_PER_TILE)],
                        magg_out.at[pl.ds(rbase, ROWS_PER_TILE)])

    @pl.when(cid == 0)
    def _dump_p():
        pltpu.sync_copy(acc.at[pl.ds(rbase, ROWS_PER_TILE)],
                        posacc_out.at[pl.ds(rbase, ROWS_PER_TILE)])


def _scatter(m2, rw, dst, z128):
    mesh = plsc.VectorSubcoreMesh(core_axis_name="c", subcore_axis_name="s")
    fn = pl.kernel(
        _scatter_body,
        out_type=[jax.ShapeDtypeStruct((NPAD, H), jnp.float32),
                  jax.ShapeDtypeStruct((NPAD, H), jnp.float32)],
        mesh=mesh,
        scratch_types=[pltpu.VMEM_SHARED((NPAD, H), jnp.float32),
                       pltpu.VMEM((2, CHUNK), jnp.int32),
                       pltpu.VMEM((2, CHUNK, H), jnp.float32),
                       pltpu.VMEM((2, 8, CHUNK), jnp.float32),
                       pltpu.SemaphoreType.DMA,
                       pltpu.SemaphoreType.DMA,
                       pltpu.SemaphoreType.DMA,
                       pltpu.SemaphoreType.DMA,
                       pltpu.SemaphoreType.DMA,
                       pltpu.SemaphoreType.DMA],
        compiler_params=pltpu.CompilerParams(needs_layout_passes=False),
    )
    return fn(m2, rw, dst, z128)


def _final_body(hp_ref, magg_ref, pacc_ref, pos_ref, wh1a_ref, wh1b_ref,
                bh1_ref, wh2_ref, bh2_ref, hout_ref, pout_ref):
    hp = hp_ref[...]
    magg = magg_ref[...]
    pacc = pacc_ref[...]
    t = _silu(jnp.dot(hp, wh1a_ref[...], preferred_element_type=jnp.float32)
              + jnp.dot(magg, wh1b_ref[...], preferred_element_type=jnp.float32)
              + bh1_ref[...])
    hout_ref[...] = hp + jnp.dot(t, wh2_ref[...], preferred_element_type=jnp.float32) + bh2_ref[...]
    lane = lax.broadcasted_iota(jnp.int32, (NB, H), 1)
    cnt = jnp.sum(jnp.where(lane == 3, pacc, 0.0), axis=1, keepdims=True)
    upd = jnp.where(lane < 3, pacc, 0.0) / jnp.maximum(cnt, 1.0)
    pout_ref[...] = pos_ref[...] + upd


def _final(hp, magg, pacc, pos128, wh1a, wh1b, bh1, wh2, bh2):
    grid = (NPAD // NB,)
    nblk = pl.BlockSpec((NB, D), lambda i: (i, 0))
    full = lambda shape: pl.BlockSpec(shape, lambda i: tuple(0 for _ in shape))
    return pl.pallas_call(
        _final_body,
        grid=grid,
        in_specs=[nblk, nblk, nblk, nblk,
                  full((D, H)), full((H, H)), full((1, H)),
                  full((H, D)), full((1, D))],
        out_specs=[nblk, nblk],
        out_shape=[jax.ShapeDtypeStruct((NPAD, D), jnp.float32),
                   jax.ShapeDtypeStruct((NPAD, D), jnp.float32)],
    )(hp, magg, pacc, pos128, wh1a, wh1b, bh1, wh2, bh2)


@jax.jit
def kernel(h, pos, edge_index, h_init, W_e1, b_e1, W_e2, b_e2, W_c1, b_c1,
           W_c2, W_h1, b_h1, W_h2, b_h2):
    h_pad = jnp.pad(h, ((0, NPAD - N), (0, 0)))
    hi_pad = jnp.pad(h_init, ((0, NPAD - N), (0, 0)))
    pos128 = jnp.pad(pos, ((0, NPAD - N), (0, D - 3)))
    posp = jnp.pad(pos, ((0, NPAD - N), (0, 0)))
    xu = jax.lax.bitcast_convert_type(posp[:, 0], jnp.uint32)
    yu = jax.lax.bitcast_convert_type(posp[:, 1], jnp.uint32)
    xy = jax.lax.bitcast_convert_type((xu & jnp.uint32(0xFFFF0000)) | (yu >> 16),
                                      jnp.float32)
    pos2 = jnp.stack([xy, posp[:, 2]], axis=1).reshape(-1)
    src = jnp.pad(edge_index[0], (0, EPAD - E))
    dst = jnp.pad(edge_index[1], (0, EPAD - E), constant_values=N)

    w1d = W_e1[:D]
    w1s = W_e1[D:2 * D]
    w1e = W_e1[2 * D:]
    b1 = b_e1.reshape(1, H)
    be2 = b_e2.reshape(1, H)
    bc1 = b_c1.reshape(1, H)
    wh1a = W_h1[:D]
    wh1b = W_h1[D:]
    bh1 = b_h1.reshape(1, H)
    bh2 = b_h2.reshape(1, D)

    hp, ha, hb = _prep(h_pad, hi_pad, w1d, w1s, b1)
    qa, qb, geom = _gather(ha, hb, pos2, src, dst)
    m2, rw = _mlp(qa, qb, geom, w1e, W_e2, be2, W_c1, bc1, W_c2)
    z128 = jnp.zeros((NPAD, H), jnp.float32)
    magg, pacc = _scatter(m2, rw, dst, z128)
    h_out, pos_out = _final(hp, magg, pacc, pos128, wh1a, wh1b, bh1, W_h2, bh2)
    return (h_out[:N], pos_out[:N, :3])


# speedup vs baseline: 9.8932x; 1.1801x over previous
import functools

import jax
import jax.numpy as jnp
from jax import lax
from jax.experimental import pallas as pl
from jax.experimental.pallas import tpu as pltpu
from jax.experimental.pallas import tpu_sc as plsc

N = 10000
E = 320000
D = 128
H = 128

NPAD = 10240
EPAD = 327680
NC = 2
NS = 16
NW = NC * NS
EW = EPAD // NW
CHUNK = 128
NCHUNK = EW // CHUNK
ROWS_PER_TILE = NPAD // NS

NB = 1024
EB = 2048
SLICES = 2
ES = EPAD // SLICES


def _silu(x):
    return x * jax.nn.sigmoid(x)


def _prep_body(h_ref, hi_ref, w1d_ref, w1s_ref, b1_ref, hp_ref, ha_ref, hb_ref):
    hp = h_ref[...] + hi_ref[...]
    hp_ref[...] = hp
    ha_ref[...] = jnp.dot(hp, w1d_ref[...], preferred_element_type=jnp.float32) + b1_ref[...]
    hb_ref[...] = jnp.dot(hp, w1s_ref[...], preferred_element_type=jnp.float32)


def _prep(h_pad, hi_pad, w1d, w1s, b1):
    grid = (NPAD // NB,)
    blk = pl.BlockSpec((NB, D), lambda i: (i, 0))
    wblk = pl.BlockSpec((D, H), lambda i: (0, 0))
    bblk = pl.BlockSpec((1, H), lambda i: (0, 0))
    return pl.pallas_call(
        _prep_body,
        grid=grid,
        in_specs=[blk, blk, wblk, wblk, bblk],
        out_specs=[blk, pl.BlockSpec((NB, H), lambda i: (i, 0)),
                   pl.BlockSpec((NB, H), lambda i: (i, 0))],
        out_shape=[jax.ShapeDtypeStruct((NPAD, D), jnp.float32),
                   jax.ShapeDtypeStruct((NPAD, H), jnp.float32),
                   jax.ShapeDtypeStruct((NPAD, H), jnp.float32)],
    )(h_pad, hi_pad, w1d, w1s, b1)


CG = 64
NSLOT = 2
UNROLL = 4
TPTG = ES // NS
NCHG = TPTG // CG


def _gather_body(ha_hbm, hb_hbm, pos_hbm, src_hbm, dst_hbm,
                 qa_out, qb_out, geom_out,
                 table_sp, sidx_v, didx_v, q_v, pos_v, geom_v, *sems):
    cid = lax.axis_index("c")
    sid = lax.axis_index("s")
    isem = sems[0:NSLOT]
    gsem = sems[NSLOT:2 * NSLOT]
    wsem = sems[2 * NSLOT:3 * NSLOT]
    gwsem = sems[3 * NSLOT:3 * NSLOT + 2]

    rbase = sid * ROWS_PER_TILE

    @pl.when(cid == 0)
    def _stage_a():
        pltpu.sync_copy(ha_hbm.at[pl.ds(rbase, ROWS_PER_TILE)],
                        table_sp.at[pl.ds(rbase, ROWS_PER_TILE)])

    @pl.when(cid == 1)
    def _stage_b():
        pltpu.sync_copy(hb_hbm.at[pl.ds(rbase, ROWS_PER_TILE)],
                        table_sp.at[pl.ds(rbase, ROWS_PER_TILE)])

    @pl.when(cid == 0)
    def _stage_pos():
        pltpu.sync_copy(pos_hbm, pos_v)
        z16 = jnp.zeros((16,), jnp.float32)
        for p in range(2):
            for c in range(3, 8):
                for g in range(128 // 16):
                    geom_v[p, c, pl.ds(g * 16, 16)] = z16

    plsc.subcore_barrier()

    def issue_idx(k, b):
        base = sid * TPTG + k * CG

        @pl.when(cid == 0)
        def _i0():
            pltpu.async_copy(src_hbm.at[pl.ds(base, CG)], sidx_v.at[b], isem[b])
            pltpu.async_copy(dst_hbm.at[pl.ds(base, CG)], didx_v.at[b], isem[b])

        @pl.when(cid == 1)
        def _i1():
            pltpu.async_copy(src_hbm.at[pl.ds(base, CG)], sidx_v.at[b], isem[b])

    def wait_idx(b):
        pltpu.make_async_copy(src_hbm.at[pl.ds(0, CG)], sidx_v.at[b], isem[b]).wait()

        @pl.when(cid == 0)
        def _w0():
            pltpu.make_async_copy(dst_hbm.at[pl.ds(0, CG)], didx_v.at[b], isem[b]).wait()

    def fire_gathers(b):
        @pl.when(cid == 0)
        def _g0():
            pltpu.async_copy(table_sp.at[didx_v.at[b]], q_v.at[b], gsem[b])

        @pl.when(cid == 1)
        def _g1():
            pltpu.async_copy(table_sp.at[sidx_v.at[b]], q_v.at[b], gsem[b])

    def wait_gathers(b):
        pltpu.make_async_copy(table_sp.at[pl.ds(0, CG)], q_v.at[b], gsem[b]).wait()

    M_HI = jnp.int32(-65536)

    def compute_geom(b, p, half):
        off = half * CG

        def grp(g, carry):
            s2 = sidx_v[b, pl.ds(g * 16, 16)] * 2
            d2 = didx_v[b, pl.ds(g * 16, 16)] * 2
            ws = plsc.bitcast(plsc.load_gather(pos_v, [s2]), jnp.int32)
            wd = plsc.bitcast(plsc.load_gather(pos_v, [d2]), jnp.int32)
            xs = plsc.bitcast(ws & M_HI, jnp.float32)
            xd = plsc.bitcast(wd & M_HI, jnp.float32)
            ys = plsc.bitcast(lax.shift_left(ws, 16), jnp.float32)
            yd = plsc.bitcast(lax.shift_left(wd, 16), jnp.float32)
            zs = plsc.load_gather(pos_v, [s2 + 1])
            zd = plsc.load_gather(pos_v, [d2 + 1])
            geom_v[p, 0, pl.ds(off + g * 16, 16)] = xd - xs
            geom_v[p, 1, pl.ds(off + g * 16, 16)] = yd - ys
            geom_v[p, 2, pl.ds(off + g * 16, 16)] = zd - zs
            return carry
        lax.fori_loop(0, CG // 16, grp, None)

    def fire_writebacks(k, b):
        base = sid * TPTG + k * CG

        @pl.when(cid == 0)
        def _w0():
            pltpu.async_copy(q_v.at[b], qa_out.at[pl.ds(base, CG)], wsem[b])

        @pl.when(cid == 1)
        def _w1():
            pltpu.async_copy(q_v.at[b], qb_out.at[pl.ds(base, CG)], wsem[b])

    def wait_writebacks(b):
        pltpu.make_async_copy(q_v.at[b], qa_out.at[pl.ds(0, CG)], wsem[b]).wait()

    def fire_geom_wb(k, p):
        base = sid * TPTG + (k - 1) * CG
        pltpu.async_copy(geom_v.at[p], geom_out.at[:, pl.ds(base, 2 * CG)], gwsem[p])

    def wait_geom_wb(p):
        pltpu.make_async_copy(geom_v.at[p], geom_out.at[:, pl.ds(0, 2 * CG)], gwsem[p]).wait()

    for b in range(NSLOT):
        issue_idx(b, b)

    def body(j, carry):
        for kk in range(UNROLL):
            k = UNROLL * j + kk
            b = kk % NSLOT
            b2 = (kk + 1) % NSLOT
            p = (kk // 2) % 2

            wait_idx(b)

            @pl.when(k >= NSLOT)
            def _free_slot():
                wait_writebacks(b)
            fire_gathers(b)

            @pl.when(cid == 0)
            def _geom():
                if kk % 2 == 0:
                    @pl.when(k >= 4)
                    def _free_geom_pair():
                        wait_geom_wb(p)
                compute_geom(b, p, kk % 2)
                if kk % 2 == 1:
                    fire_geom_wb(k, p)

            @pl.when(k >= NSLOT - 1)
            def _finish_old():
                jj = k - (NSLOT - 1)
                wait_gathers(b2)
                fire_writebacks(jj, b2)

                @pl.when(jj + NSLOT < NCHG)
                def _prefetch_idx():
                    issue_idx(jj + NSLOT, b2)
        return carry

    lax.fori_loop(0, NCHG // UNROLL, body, None)
    for t in range(NCHG - (NSLOT - 1), NCHG):
        b2 = t % NSLOT
        wait_gathers(b2)
        fire_writebacks(t, b2)
    for b in range(NSLOT):
        wait_writebacks(b)

    @pl.when(cid == 0)
    def _drain_geom():
        for p in range(2):
            wait_geom_wb(p)


def _gather(ha, hb, pos2, src, dst):
    mesh = plsc.VectorSubcoreMesh(core_axis_name="c", subcore_axis_name="s")
    fn = pl.kernel(
        _gather_body,
        out_type=[jax.ShapeDtypeStruct((ES, H), jnp.float32),
                  jax.ShapeDtypeStruct((ES, H), jnp.float32),
                  jax.ShapeDtypeStruct((8, ES), jnp.float32)],
        mesh=mesh,
        scratch_types=[pltpu.VMEM_SHARED((NPAD, H), jnp.float32),
                       pltpu.VMEM((NSLOT, CG), jnp.int32),
                       pltpu.VMEM((NSLOT, CG), jnp.int32),
                       pltpu.VMEM((NSLOT, CG, H), jnp.float32),
                       pltpu.VMEM((NPAD * 2,), jnp.float32),
                       pltpu.VMEM((2, 8, 2 * CG), jnp.float32)]
                      + [pltpu.SemaphoreType.DMA] * (3 * NSLOT + 2),
        compiler_params=pltpu.CompilerParams(needs_layout_passes=False),
    )
    return fn(ha, hb, pos2, src, dst)


def _mlp_body(qa_ref, qb_ref, geom_ref, w1e_ref, we2_ref, be2_ref,
              wc1_ref, bc1_ref, wc2_ref, m2_ref, rw_ref):
    rel = jnp.transpose(geom_ref[...])
    dist2 = jnp.sum(rel * rel, axis=1, keepdims=True)
    m1 = _silu(qa_ref[...] + qb_ref[...] + dist2 * w1e_ref[...])
    m2 = _silu(jnp.dot(m1, we2_ref[...], preferred_element_type=jnp.float32) + be2_ref[...])
    cw = jnp.dot(_silu(jnp.dot(m2, wc1_ref[...], preferred_element_type=jnp.float32) + bc1_ref[...]),
                 wc2_ref[...], preferred_element_type=jnp.float32)
    m2_ref[...] = m2
    sub = lax.broadcasted_iota(jnp.int32, (8, EB), 0)
    rw_ref[...] = jnp.where(sub == 3, 1.0, geom_ref[...] * jnp.transpose(cw))


def _mlp(qa, qb, geom, w1e, we2, be2, wc1, bc1, wc2):
    grid = (ES // EB,)
    eblk = pl.BlockSpec((EB, H), lambda i: (i, 0))
    gblk = pl.BlockSpec((8, EB), lambda i: (0, i))
    full = lambda shape: pl.BlockSpec(shape, lambda i: tuple(0 for _ in shape))
    return pl.pallas_call(
        _mlp_body,
        grid=grid,
        in_specs=[eblk, eblk, gblk,
                  full((1, H)), full((H, H)), full((1, H)),
                  full((H, H)), full((1, H)), full((H, 1))],
        out_specs=[eblk, gblk],
        out_shape=[jax.ShapeDtypeStruct((ES, H), jnp.float32),
                   jax.ShapeDtypeStruct((8, ES), jnp.float32)],
    )(qa, qb, geom, w1e, we2, be2, wc1, bc1, wc2)


TPT = ES // NS
NCH2 = TPT // CHUNK


def _scatter_body(m2_hbm, rw_hbm, dst_hbm, z128_hbm,
                  magg_out, posacc_out,
                  acc, didx_v, m2_v, rw_tv,
                  isem0, isem1, rsem0, rsem1, asem0, asem1):
    rw128_v = m2_v
    cid = lax.axis_index("c")
    sid = lax.axis_index("s")
    rbase = sid * ROWS_PER_TILE
    isem = (isem0, isem1)
    rsem = (rsem0, rsem1)
    asem = (asem0, asem1)
    pltpu.sync_copy(z128_hbm.at[pl.ds(rbase, ROWS_PER_TILE)],
                    acc.at[pl.ds(rbase, ROWS_PER_TILE)])
    plsc.subcore_barrier()

    def issue_idx(k, b):
        base = sid * TPT + k * CHUNK
        pltpu.async_copy(dst_hbm.at[pl.ds(base, CHUNK)], didx_v.at[b], isem[b])

    def wait_idx(b):
        pltpu.make_async_copy(dst_hbm.at[pl.ds(0, CHUNK)], didx_v.at[b], isem[b]).wait()

    @pl.when(cid == 1)
    def _m2_loop():
        def issue_read(k, b):
            base = sid * TPT + k * CHUNK
            pltpu.async_copy(m2_hbm.at[pl.ds(base, CHUNK)], m2_v.at[b], rsem[b])

        issue_idx(0, 0); issue_read(0, 0)
        issue_idx(1, 1); issue_read(1, 1)

        def body(j, carry):
            for b in range(2):
                k = 2 * j + b
                wait_idx(b)
                pltpu.make_async_copy(m2_hbm.at[pl.ds(0, CHUNK)], m2_v.at[b], rsem[b]).wait()
                add = pltpu.async_copy(m2_v.at[b], acc.at[didx_v.at[b]], asem[b], add=True)
                add.wait()

                @pl.when(k + 2 < NCH2)
                def _prefetch():
                    issue_idx(k + 2, b)
                    issue_read(k + 2, b)
            return carry
        lax.fori_loop(0, NCH2 // 2, body, None)

    @pl.when(cid == 0)
    def _rw_loop():
        pltpu.sync_copy(z128_hbm.at[pl.ds(0, CHUNK)], rw128_v.at[0])
        pltpu.sync_copy(z128_hbm.at[pl.ds(0, CHUNK)], rw128_v.at[1])
        lane16 = lax.broadcasted_iota(jnp.int32, (16,), 0)

        def issue_read(k, b):
            base = sid * TPT + k * CHUNK
            pltpu.async_copy(rw_hbm.at[:, pl.ds(base, CHUNK)], rw_tv.at[b], rsem[b])

        issue_idx(0, 0); issue_read(0, 0)
        issue_idx(1, 1); issue_read(1, 1)

        def body(j, carry):
            for b in range(2):
                k = 2 * j + b
                wait_idx(b)
                pltpu.make_async_copy(rw_hbm.at[:, pl.ds(0, CHUNK)], rw_tv.at[b], rsem[b]).wait()

                def grp(g, c2):
                    e16 = g * 16 + lane16
                    for c in range(4):
                        plsc.store_scatter(rw128_v.at[b],
                                           [e16, jnp.full((16,), c, jnp.int32)],
                                           rw_tv[b, c, pl.ds(g * 16, 16)])
                    return c2
                lax.fori_loop(0, CHUNK // 16, grp, None)
                add = pltpu.async_copy(rw128_v.at[b], acc.at[didx_v.at[b]], asem[b], add=True)
                add.wait()

                @pl.when(k + 2 < NCH2)
                def _prefetch():
                    issue_idx(k + 2, b)
                    issue_read(k + 2, b)
            return carry
        lax.fori_loop(0, NCH2 // 2, body, None)

    plsc.subcore_barrier()

    @pl.when(cid == 1)
    def _dump_m():
        pltpu.sync_copy(acc.at[pl.ds(rbase, ROWS_PER_TILE)],
                        magg_out.at[pl.ds(rbase, ROWS_PER_TILE)])

    @pl.when(cid == 0)
    def _dump_p():
        pltpu.sync_copy(acc.at[pl.ds(rbase, ROWS_PER_TILE)],
                        posacc_out.at[pl.ds(rbase, ROWS_PER_TILE)])


def _scatter(m2, rw, dst, z128):
    mesh = plsc.VectorSubcoreMesh(core_axis_name="c", subcore_axis_name="s")
    fn = pl.kernel(
        _scatter_body,
        out_type=[jax.ShapeDtypeStruct((NPAD, H), jnp.float32),
                  jax.ShapeDtypeStruct((NPAD, H), jnp.float32)],
        mesh=mesh,
        scratch_types=[pltpu.VMEM_SHARED((NPAD, H), jnp.float32),
                       pltpu.VMEM((2, CHUNK), jnp.int32),
                       pltpu.VMEM((2, CHUNK, H), jnp.float32),
                       pltpu.VMEM((2, 8, CHUNK), jnp.float32),
                       pltpu.SemaphoreType.DMA,
                       pltpu.SemaphoreType.DMA,
                       pltpu.SemaphoreType.DMA,
                       pltpu.SemaphoreType.DMA,
                       pltpu.SemaphoreType.DMA,
                       pltpu.SemaphoreType.DMA],
        compiler_params=pltpu.CompilerParams(needs_layout_passes=False),
    )
    return fn(m2, rw, dst, z128)


def _final_body(hp_ref, magg_ref, magg1_ref, pacc_ref, pacc1_ref, pos_ref,
                wh1a_ref, wh1b_ref, bh1_ref, wh2_ref, bh2_ref, hout_ref, pout_ref):
    hp = hp_ref[...]
    magg = magg_ref[...] + magg1_ref[...]
    pacc = pacc_ref[...] + pacc1_ref[...]
    t = _silu(jnp.dot(hp, wh1a_ref[...], preferred_element_type=jnp.float32)
              + jnp.dot(magg, wh1b_ref[...], preferred_element_type=jnp.float32)
              + bh1_ref[...])
    hout_ref[...] = hp + jnp.dot(t, wh2_ref[...], preferred_element_type=jnp.float32) + bh2_ref[...]
    lane = lax.broadcasted_iota(jnp.int32, (NB, H), 1)
    cnt = jnp.sum(jnp.where(lane == 3, pacc, 0.0), axis=1, keepdims=True)
    upd = jnp.where(lane < 3, pacc, 0.0) / jnp.maximum(cnt, 1.0)
    pout_ref[...] = pos_ref[...] + upd


def _final(hp, magg0, magg1, pacc0, pacc1, pos128, wh1a, wh1b, bh1, wh2, bh2):
    grid = (NPAD // NB,)
    nblk = pl.BlockSpec((NB, D), lambda i: (i, 0))
    full = lambda shape: pl.BlockSpec(shape, lambda i: tuple(0 for _ in shape))
    return pl.pallas_call(
        _final_body,
        grid=grid,
        in_specs=[nblk, nblk, nblk, nblk, nblk, nblk,
                  full((D, H)), full((H, H)), full((1, H)),
                  full((H, D)), full((1, D))],
        out_specs=[nblk, nblk],
        out_shape=[jax.ShapeDtypeStruct((NPAD, D), jnp.float32),
                   jax.ShapeDtypeStruct((NPAD, D), jnp.float32)],
    )(hp, magg0, magg1, pacc0, pacc1, pos128, wh1a, wh1b, bh1, wh2, bh2)


@jax.jit
def kernel(h, pos, edge_index, h_init, W_e1, b_e1, W_e2, b_e2, W_c1, b_c1,
           W_c2, W_h1, b_h1, W_h2, b_h2):
    h_pad = jnp.pad(h, ((0, NPAD - N), (0, 0)))
    hi_pad = jnp.pad(h_init, ((0, NPAD - N), (0, 0)))
    pos128 = jnp.pad(pos, ((0, NPAD - N), (0, D - 3)))
    posp = jnp.pad(pos, ((0, NPAD - N), (0, 0)))
    xu = jax.lax.bitcast_convert_type(posp[:, 0], jnp.uint32)
    yu = jax.lax.bitcast_convert_type(posp[:, 1], jnp.uint32)
    xy = jax.lax.bitcast_convert_type((xu & jnp.uint32(0xFFFF0000)) | (yu >> 16),
                                      jnp.float32)
    pos2 = jnp.stack([xy, posp[:, 2]], axis=1).reshape(-1)
    src = jnp.pad(edge_index[0], (0, EPAD - E))
    dst = jnp.pad(edge_index[1], (0, EPAD - E), constant_values=N)

    w1d = W_e1[:D]
    w1s = W_e1[D:2 * D]
    w1e = W_e1[2 * D:]
    b1 = b_e1.reshape(1, H)
    be2 = b_e2.reshape(1, H)
    bc1 = b_c1.reshape(1, H)
    wh1a = W_h1[:D]
    wh1b = W_h1[D:]
    bh1 = b_h1.reshape(1, H)
    bh2 = b_h2.reshape(1, D)

    hp, ha, hb = _prep(h_pad, hi_pad, w1d, w1s, b1)
    z128 = jnp.zeros((NPAD, H), jnp.float32)
    maggs, paccs = [], []
    for sl in range(SLICES):
        src_s = src[sl * ES:(sl + 1) * ES]
        dst_s = dst[sl * ES:(sl + 1) * ES]
        qa, qb, geom = _gather(ha, hb, pos2, src_s, dst_s)
        m2, rw = _mlp(qa, qb, geom, w1e, W_e2, be2, W_c1, bc1, W_c2)
        magg_s, pacc_s = _scatter(m2, rw, dst_s, z128)
        maggs.append(magg_s)
        paccs.append(pacc_s)
    h_out, pos_out = _final(hp, maggs[0], maggs[1], paccs[0], paccs[1],
                            pos128, wh1a, wh1b, bh1, W_h2, bh2)
    return (h_out[:N], pos_out[:N, :3])
